# baseline (speedup 1.0000x reference)
import functools

import jax
import jax.numpy as jnp
from jax import lax
from jax.experimental import pallas as pl
from jax.experimental.pallas import tpu as pltpu
from jax.experimental.pallas import tpu_sc as plsc

N = 10000
NPAD = 10112
RPT = NPAD // 16
E = 160000
G = 128
NGRP = E // G
F_IN = 256
H = 128
O = 64
C = 16

NC = 2
NS = 16
NW = NC * NS


def _sc_mesh():
    return plsc.VectorSubcoreMesh(core_axis_name="c", subcore_axis_name="s")


def _zero_vmem2d(buf, rows, cols):
    zero = jnp.zeros((16,), jnp.float32)

    def zb(i, _):
        r = i // (cols // 16)
        c = i % (cols // 16)
        buf[r, pl.ds(c * 16, 16)] = zero
        return 0

    lax.fori_loop(0, rows * (cols // 16), zb, 0)


def _zero_acc_slice(zbuf, acc_sh, sid):
    nfull = RPT // 64
    for t in range(nfull):
        pltpu.sync_copy(zbuf, acc_sh.at[pl.ds(sid * RPT + t * 64, 64)])
    rem = RPT - nfull * 64
    if rem:
        pltpu.sync_copy(zbuf.at[pl.ds(0, rem)],
                        acc_sh.at[pl.ds(sid * RPT + nfull * 64, rem)])


def _sc_degree(dst):
    ept = E // NW
    full = ept // 16
    tail = ept - full * 16

    @functools.partial(
        pl.kernel,
        mesh=_sc_mesh(),
        compiler_params=pltpu.CompilerParams(needs_layout_passes=False),
        out_type=jax.ShapeDtypeStruct((NW, NPAD), jnp.float32),
        scratch_types=[
            pltpu.VMEM((NPAD,), jnp.float32),
            pltpu.VMEM((ept + 16,), jnp.int32),
        ],
    )
    def k(dst_hbm, out_hbm, hist, idx):
        cid = lax.axis_index("c")
        sid = lax.axis_index("s")
        wid = sid * NC + cid

        zero = jnp.zeros((16,), jnp.float32)

        def zbody(i, _):
            hist[pl.ds(i * 16, 16)] = zero
            return 0

        lax.fori_loop(0, NPAD // 16, zbody, 0)

        pltpu.sync_copy(dst_hbm.at[pl.ds(wid * ept, ept)],
                        idx.at[pl.ds(0, ept)])

        ones = jnp.ones((16,), jnp.float32)

        def hbody(i, _):
            v = idx[pl.ds(i * 16, 16)]
            plsc.addupdate_scatter(hist, [v], ones)
            return 0

        lax.fori_loop(0, full, hbody, 0)

        if tail:
            lanes = lax.iota(jnp.int32, 16)
            tmask = lanes < tail
            v = idx[pl.ds(full * 16, 16)]
            v = jnp.where(tmask, v, 0)
            plsc.addupdate_scatter(hist, [v], ones, mask=tmask)

        pltpu.sync_copy(hist, out_hbm.at[wid])

    return k(dst)


OWN = 640
NPAD2 = 16 * OWN
ACC_ROWS = OWN + 8
ST_CAP = 160
EB = 1024
K_BUF = 2


def _drain(hs, acc_flat, st_src, st_dstl, rows, gsem):
    pltpu.async_copy(hs.at[st_src.at[pl.ds(0, G)]], rows, gsem).wait()
    iota = lax.iota(jnp.int32, 16)

    def edge(e, _):
        base = plsc.load_gather(
            st_dstl, [jnp.broadcast_to(e, (16,)).astype(jnp.int32)])
        addr = base * H + iota
        for j in range(H // 16):
            plsc.addupdate_scatter(acc_flat, [addr + j * 16],
                                   rows[e, pl.ds(j * 16, 16)])
        return 0

    lax.fori_loop(0, G, edge, 0)


def _owner_scan(hs, out_flat, src_h, dst_h, src_b, dst_b, st_src, st_dstl,
                rows, acc_flat, wpr, gsem, sid, eoff, n_edges):
    iota = lax.iota(jnp.int32, 16)
    zero = jnp.zeros((16,), jnp.float32)

    def zb(i, _):
        acc_flat[pl.ds(i * 16, 16)] = zero
        return 0

    lax.fori_loop(0, (ACC_ROWS * H) // 16, zb, 0)
    wpr[0] = 0

    my_base = sid * OWN

    def subchunk(c, _):
        s16 = src_b[pl.ds(c * 16, 16)]
        d16 = dst_b[pl.ds(c * 16, 16)]
        own16 = jnp.right_shift(d16 * 52429, 25)
        m = own16 == sid
        wp = wpr[0]
        plsc.store_compressed(st_src.at[pl.ds(wp, 16)], s16, mask=m)
        plsc.store_compressed(st_dstl.at[pl.ds(wp, 16)], d16 - my_base,
                              mask=m)
        cnt = lax.reduce_max(plsc.all_reduce_population_count(m), axes=(0,))
        wp = wp + cnt

        @pl.when(wp >= G)
        def _():
            _drain(hs, acc_flat, st_src, st_dstl, rows, gsem)
            t0 = st_src[pl.ds(G, 16)]
            t1 = st_dstl[pl.ds(G, 16)]
            st_src[pl.ds(0, 16)] = t0
            st_dstl[pl.ds(0, 16)] = t1
            wpr[0] = wp - G

        @pl.when(wp < G)
        def _():
            wpr[0] = wp

        return 0

    n_full = n_edges // EB
    tail = n_edges - n_full * EB

    def load_chunk(l, nb):
        pltpu.sync_copy(src_h.at[pl.ds(eoff + l * EB, nb)],
                        src_b.at[pl.ds(0, nb)])
        pltpu.sync_copy(dst_h.at[pl.ds(eoff + l * EB, nb)],
                        dst_b.at[pl.ds(0, nb)])
        lax.fori_loop(0, nb // 16, subchunk, 0)

    def chunk(l, _):
        load_chunk(l, EB)
        return 0

    lax.fori_loop(0, n_full, chunk, 0)
    if tail:
        load_chunk(n_full, tail)

    wp = wpr[0]
    for p in range(G // 16):
        idx16 = wp + p * 16 + iota
        pm = idx16 < G
        plsc.store_scatter(st_src, [idx16], jnp.zeros((16,), jnp.int32),
                           mask=pm)
        plsc.store_scatter(st_dstl, [idx16],
                           jnp.full((16,), OWN, jnp.int32), mask=pm)
    _drain(hs, acc_flat, st_src, st_dstl, rows, gsem)

    pltpu.sync_copy(acc_flat.at[pl.ds(0, OWN * H)],
                    out_flat.at[pl.ds(sid * (OWN * H), OWN * H)])


def _prop_scratch():
    return [
        pltpu.VMEM((EB,), jnp.int32),
        pltpu.VMEM((EB,), jnp.int32),
        pltpu.VMEM((ST_CAP,), jnp.int32),
        pltpu.VMEM((ST_CAP,), jnp.int32),
        pltpu.VMEM((G, H), jnp.float32),
        pltpu.VMEM((ACC_ROWS * H,), jnp.float32),
        pltpu.SMEM((1,), jnp.int32),
        pltpu.SemaphoreType.DMA,
    ]


def _sc_prop1(hsA, hsB, src, dst):
    @functools.partial(
        pl.kernel,
        mesh=_sc_mesh(),
        compiler_params=pltpu.CompilerParams(needs_layout_passes=False),
        out_type=(jax.ShapeDtypeStruct((NPAD2 * H,), jnp.float32),
                  jax.ShapeDtypeStruct((NPAD2 * H,), jnp.float32)),
        scratch_types=_prop_scratch(),
    )
    def k(hsA_h, hsB_h, src_h, dst_h, outA, outB, src_b, dst_b, st_src,
          st_dstl, rows, acc_flat, wpr, gsem):
        cid = lax.axis_index("c")
        sid = lax.axis_index("s")

        @pl.when(cid == 0)
        def _():
            _owner_scan(hsA_h, outA, src_h, dst_h, src_b, dst_b, st_src,
                        st_dstl, rows, acc_flat, wpr, gsem, sid, 0, E)

        @pl.when(cid == 1)
        def _():
            _owner_scan(hsB_h, outB, src_h, dst_h, src_b, dst_b, st_src,
                        st_dstl, rows, acc_flat, wpr, gsem, sid, 0, E)

    return k(hsA, hsB, src, dst)


def _sc_prop2(hs2, src, dst):
    epc = E // NC

    @functools.partial(
        pl.kernel,
        mesh=_sc_mesh(),
        compiler_params=pltpu.CompilerParams(needs_layout_passes=False),
        out_type=(jax.ShapeDtypeStruct((NPAD2 * H,), jnp.float32),
                  jax.ShapeDtypeStruct((NPAD2 * H,), jnp.float32)),
        scratch_types=_prop_scratch(),
    )
    def k(hs_h, src_h, dst_h, out0, out1, src_b, dst_b, st_src,
          st_dstl, rows, acc_flat, wpr, gsem):
        cid = lax.axis_index("c")
        sid = lax.axis_index("s")

        @pl.when(cid == 0)
        def _():
            _owner_scan(hs_h, out0, src_h, dst_h, src_b, dst_b, st_src,
                        st_dstl, rows, acc_flat, wpr, gsem, sid, 0, epc)

        @pl.when(cid == 1)
        def _():
            _owner_scan(hs_h, out1, src_h, dst_h, src_b, dst_b, st_src,
                        st_dstl, rows, acc_flat, wpr, gsem, sid, epc, epc)

    return k(hs2, src, dst)


def _sc_edge_dot(x, srcg, dstg):
    e2 = srcg.shape[0] * G
    ngrp = srcg.shape[0]
    base_g = ngrp // NW
    extra = ngrp - base_g * NW

    @functools.partial(
        pl.kernel,
        mesh=_sc_mesh(),
        compiler_params=pltpu.CompilerParams(needs_layout_passes=False,
                                             use_tc_tiling_on_sc=False),
        out_type=jax.ShapeDtypeStruct((e2,), jnp.float32),
        scratch_types=[
            pltpu.VMEM((G,), jnp.int32),
            pltpu.VMEM((G,), jnp.int32),
            pltpu.VMEM((G, O), jnp.float32),
            pltpu.VMEM((G, O), jnp.float32),
            pltpu.VMEM((G,), jnp.float32),
            pltpu.SemaphoreType.DMA,
            pltpu.SemaphoreType.DMA,
        ],
    )
    def k(x_h, srcg_h, dstg_h, res, si, di, xs, xd, out_v, sem0, sem1):
        cid = lax.axis_index("c")
        sid = lax.axis_index("s")
        wid = sid * NC + cid

        zero = jnp.zeros((16,), jnp.float32)

        def do_group(g):
            pltpu.sync_copy(srcg_h.at[g], si)
            pltpu.sync_copy(dstg_h.at[g], di)
            cp0 = pltpu.async_copy(x_h.at[si], xs, sem0)
            cp1 = pltpu.async_copy(x_h.at[di], xd, sem1)
            cp0.wait()
            cp1.wait()

            for z in range(G // 16):
                out_v[pl.ds(z * 16, 16)] = zero

            def edge(e, _):
                p = xs[e, pl.ds(0, 16)] * xd[e, pl.ds(0, 16)]
                for j in range(1, O // 16):
                    p = p + xs[e, pl.ds(j * 16, 16)] * xd[e, pl.ds(j * 16, 16)]
                eidx = jnp.broadcast_to(e, (16,)).astype(jnp.int32)
                plsc.addupdate_scatter(out_v, [eidx], p)
                return 0

            lax.fori_loop(0, G, edge, 0)
            pltpu.sync_copy(out_v, res.at[pl.ds(g * G, G)])

        def body(k_, _):
            do_group(wid + NW * k_)
            return 0

        lax.fori_loop(0, base_g, body, 0)

        @pl.when(wid < extra)
        def _():
            do_group(wid + NW * base_g)

    return k(x, srcg, dstg)


def _tc_dinv(part):
    part3 = part.reshape(NW, NPAD // 128, 128)

    def body(p_ref, o_ref):
        deg = jnp.sum(p_ref[...], axis=0) + 1.0
        o_ref[...] = lax.rsqrt(deg)

    out = pl.pallas_call(
        body,
        out_shape=jax.ShapeDtypeStruct((NPAD // 128, 128), jnp.float32),
    )(part3)
    return out.reshape(NPAD)


def _tc_mm_scale(x, w, dinv2, block_rows=2000):
    m, kdim = x.shape
    n = w.shape[1]

    def body(x_ref, w_ref, d_ref, o_ref):
        o_ref[...] = jnp.dot(x_ref[...], w_ref[...],
                             preferred_element_type=jnp.float32) * d_ref[...]

    return pl.pallas_call(
        body,
        grid=(m // block_rows,),
        in_specs=[
            pl.BlockSpec((block_rows, kdim), lambda i: (i, 0)),
            pl.BlockSpec((kdim, n), lambda i: (0, 0)),
            pl.BlockSpec((block_rows, 1), lambda i: (i, 0)),
        ],
        out_specs=pl.BlockSpec((block_rows, n), lambda i: (i, 0)),
        out_shape=jax.ShapeDtypeStruct((m, n), jnp.float32),
    )(x, w, dinv2)


def _tc_layer2(accA, hsA, accB, hsB, dinv2, b1, W2, block_rows=2000):
    m = accA.shape[0]

    def body(aA, hA, aB, hB, d_ref, b_ref, w_ref, o_ref):
        d = d_ref[...]
        tA = jax.nn.relu(d * (aA[...] + hA[...]) + b_ref[...])
        tB = jax.nn.relu(d * (aB[...] + hB[...]) + b_ref[...])
        oA = jnp.dot(tA, w_ref[...], preferred_element_type=jnp.float32) * d
        oB = jnp.dot(tB, w_ref[...], preferred_element_type=jnp.float32) * d
        o_ref[...] = jnp.concatenate([oA, oB], axis=1)

    return pl.pallas_call(
        body,
        grid=(m // block_rows,),
        in_specs=[
            pl.BlockSpec((block_rows, H), lambda i: (i, 0)),
            pl.BlockSpec((block_rows, H), lambda i: (i, 0)),
            pl.BlockSpec((block_rows, H), lambda i: (i, 0)),
            pl.BlockSpec((block_rows, H), lambda i: (i, 0)),
            pl.BlockSpec((block_rows, 1), lambda i: (i, 0)),
            pl.BlockSpec((1, H), lambda i: (0, 0)),
            pl.BlockSpec((H, O), lambda i: (0, 0)),
        ],
        out_specs=pl.BlockSpec((block_rows, 2 * O), lambda i: (i, 0)),
        out_shape=jax.ShapeDtypeStruct((m, 2 * O), jnp.float32),
    )(accA, hsA, accB, hsB, dinv2, b1.reshape(1, H), W2)


def _tc_final(acc0, acc1, hs2, dinv2, b2, Wm, bm, Wa, ba, block_rows=2000):
    m = hs2.shape[0]

    def body(a0, a1, h_ref, d_ref, b2_ref, wm_ref, bm_ref, wa_ref, ba_ref,
             x_ref, att_ref):
        d = d_ref[...]
        xc = d * (a0[...] + a1[...] + h_ref[...]) + b2_ref[...]
        x_ref[...] = jnp.dot(xc, wm_ref[...],
                             preferred_element_type=jnp.float32) + bm_ref[...]
        x2 = xc[:, O:]
        t = jnp.dot(x2, wa_ref[...],
                    preferred_element_type=jnp.float32) + ba_ref[...]
        tm = jnp.max(t, axis=1, keepdims=True)
        tt = t - tm
        att_ref[...] = tt - jnp.log(jnp.sum(jnp.exp(tt), axis=1,
                                            keepdims=True))

    b22 = jnp.concatenate([b2, b2]).reshape(1, 2 * O)
    return pl.pallas_call(
        body,
        grid=(m // block_rows,),
        in_specs=[
            pl.BlockSpec((block_rows, 2 * O), lambda i: (i, 0)),
            pl.BlockSpec((block_rows, 2 * O), lambda i: (i, 0)),
            pl.BlockSpec((block_rows, 2 * O), lambda i: (i, 0)),
            pl.BlockSpec((block_rows, 1), lambda i: (i, 0)),
            pl.BlockSpec((1, 2 * O), lambda i: (0, 0)),
            pl.BlockSpec((2 * O, O), lambda i: (0, 0)),
            pl.BlockSpec((1, O), lambda i: (0, 0)),
            pl.BlockSpec((O, C), lambda i: (0, 0)),
            pl.BlockSpec((1, C), lambda i: (0, 0)),
        ],
        out_specs=[
            pl.BlockSpec((block_rows, O), lambda i: (i, 0)),
            pl.BlockSpec((block_rows, C), lambda i: (i, 0)),
        ],
        out_shape=[
            jax.ShapeDtypeStruct((m, O), jnp.float32),
            jax.ShapeDtypeStruct((m, C), jnp.float32),
        ],
    )(acc0, acc1, hs2, dinv2, b22, Wm, bm.reshape(1, O), Wa,
      ba.reshape(1, C))


def kernel(x_A, x_B, train_pos_edge_index, pos_edge_index, neg_edge_index,
           W1, b1, W2, b2, Wm, bm, Wa, ba):
    src = train_pos_edge_index[0]
    dst = train_pos_edge_index[1]

    part = _sc_degree(dst)
    dinv = _tc_dinv(part)[:N]
    dinv2 = dinv[:, None]

    hsA = _tc_mm_scale(x_A, W1, dinv2)
    hsB = _tc_mm_scale(x_B, W1, dinv2)

    accA, accB = _sc_prop1(hsA, hsB, src, dst)
    accA = accA.reshape(NPAD2, H)[:N]
    accB = accB.reshape(NPAD2, H)[:N]

    hs2 = _tc_layer2(accA, hsA, accB, hsB, dinv2, b1, W2)

    acc0, acc1 = _sc_prop2(hs2, src, dst)
    acc0 = acc0.reshape(NPAD2, H)[:N]
    acc1 = acc1.reshape(NPAD2, H)[:N]

    x, att = _tc_final(acc0, acc1, hs2, dinv2, b2, Wm, bm, Wa, ba)

    tot = jnp.concatenate([pos_edge_index, neg_edge_index], axis=-1)
    e2 = tot.shape[1]
    res = _sc_edge_dot(x, tot[0].reshape(e2 // G, G), tot[1].reshape(e2 // G, G))
    return res, att


# speedup vs baseline: 1.0344x; 1.0344x over previous
import functools

import jax
import jax.numpy as jnp
from jax import lax
from jax.experimental import pallas as pl
from jax.experimental.pallas import tpu as pltpu
from jax.experimental.pallas import tpu_sc as plsc

N = 10000
NPAD = 10112
RPT = NPAD // 16
E = 160000
G = 128
NGRP = E // G
F_IN = 256
H = 128
O = 64
C = 16

NC = 2
NS = 16
NW = NC * NS


def _sc_mesh():
    return plsc.VectorSubcoreMesh(core_axis_name="c", subcore_axis_name="s")


def _zero_vmem2d(buf, rows, cols):
    zero = jnp.zeros((16,), jnp.float32)

    def zb(i, _):
        r = i // (cols // 16)
        c = i % (cols // 16)
        buf[r, pl.ds(c * 16, 16)] = zero
        return 0

    lax.fori_loop(0, rows * (cols // 16), zb, 0)


def _zero_acc_slice(zbuf, acc_sh, sid):
    nfull = RPT // 64
    for t in range(nfull):
        pltpu.sync_copy(zbuf, acc_sh.at[pl.ds(sid * RPT + t * 64, 64)])
    rem = RPT - nfull * 64
    if rem:
        pltpu.sync_copy(zbuf.at[pl.ds(0, rem)],
                        acc_sh.at[pl.ds(sid * RPT + nfull * 64, rem)])


def _sc_degree(dst):
    ept = E // NW
    full = ept // 16
    tail = ept - full * 16

    @functools.partial(
        pl.kernel,
        mesh=_sc_mesh(),
        compiler_params=pltpu.CompilerParams(needs_layout_passes=False),
        out_type=jax.ShapeDtypeStruct((NW, NPAD), jnp.float32),
        scratch_types=[
            pltpu.VMEM((NPAD,), jnp.float32),
            pltpu.VMEM((ept + 16,), jnp.int32),
        ],
    )
    def k(dst_hbm, out_hbm, hist, idx):
        cid = lax.axis_index("c")
        sid = lax.axis_index("s")
        wid = sid * NC + cid

        zero = jnp.zeros((16,), jnp.float32)

        def zbody(i, _):
            hist[pl.ds(i * 16, 16)] = zero
            return 0

        lax.fori_loop(0, NPAD // 16, zbody, 0)

        pltpu.sync_copy(dst_hbm.at[pl.ds(wid * ept, ept)],
                        idx.at[pl.ds(0, ept)])

        ones = jnp.ones((16,), jnp.float32)

        def hbody(i, _):
            v = idx[pl.ds(i * 16, 16)]
            plsc.addupdate_scatter(hist, [v], ones)
            return 0

        lax.fori_loop(0, full, hbody, 0)

        if tail:
            lanes = lax.iota(jnp.int32, 16)
            tmask = lanes < tail
            v = idx[pl.ds(full * 16, 16)]
            v = jnp.where(tmask, v, 0)
            plsc.addupdate_scatter(hist, [v], ones, mask=tmask)

        pltpu.sync_copy(hist, out_hbm.at[wid])

    return k(dst)


OWN = 640
NPAD2 = 16 * OWN
ACC_ROWS = OWN + 8
ST_CAP = 160
EB = 1024
K_BUF = 2


def _drain(hs, acc_flat, st_src, st_dstl, rows, gsem):
    pltpu.async_copy(hs.at[st_src.at[pl.ds(0, G)]], rows, gsem).wait()
    iota = lax.iota(jnp.int32, 16)

    def edge(e, _):
        base = plsc.load_gather(
            st_dstl, [jnp.broadcast_to(e, (16,)).astype(jnp.int32)])
        addr = base * H + iota
        for j in range(H // 16):
            plsc.addupdate_scatter(acc_flat, [addr + j * 16],
                                   rows[e, pl.ds(j * 16, 16)])
        return 0

    lax.fori_loop(0, G, edge, 0)


def _owner_scan(hs, out_flat, src_h, dst_h, src_b, dst_b, st_src, st_dstl,
                rows, acc_flat, wpr, gsem, sid, eoff, n_edges):
    iota = lax.iota(jnp.int32, 16)
    zero = jnp.zeros((16,), jnp.float32)

    def zb(i, _):
        acc_flat[pl.ds(i * 16, 16)] = zero
        return 0

    lax.fori_loop(0, (ACC_ROWS * H) // 16, zb, 0)
    wpr[0] = 0

    my_base = sid * OWN

    def subchunk(c):
        s16 = src_b[pl.ds(c * 16, 16)]
        d16 = dst_b[pl.ds(c * 16, 16)]
        own16 = jnp.right_shift(d16 * 52429, 25)
        m = own16 == sid
        wp = wpr[0]
        plsc.store_compressed(st_src.at[pl.ds(wp, 16)], s16, mask=m)
        plsc.store_compressed(st_dstl.at[pl.ds(wp, 16)], d16 - my_base,
                              mask=m)
        wp = wp + lax.reduce_sum(m.astype(jnp.int32), axes=(0,))

        @pl.when(wp >= G)
        def _():
            _drain(hs, acc_flat, st_src, st_dstl, rows, gsem)
            t0 = st_src[pl.ds(G, 16)]
            t1 = st_dstl[pl.ds(G, 16)]
            st_src[pl.ds(0, 16)] = t0
            st_dstl[pl.ds(0, 16)] = t1
            wpr[0] = wp - G

        @pl.when(wp < G)
        def _():
            wpr[0] = wp

    UNROLL = 4

    def subchunk4(i, _):
        for u in range(UNROLL):
            subchunk(i * UNROLL + u)
        return 0

    n_full = n_edges // EB
    tail = n_edges - n_full * EB

    def load_chunk(l, nb):
        pltpu.sync_copy(src_h.at[pl.ds(eoff + l * EB, nb)],
                        src_b.at[pl.ds(0, nb)])
        pltpu.sync_copy(dst_h.at[pl.ds(eoff + l * EB, nb)],
                        dst_b.at[pl.ds(0, nb)])
        lax.fori_loop(0, nb // (16 * UNROLL), subchunk4, 0)

    def chunk(l, _):
        load_chunk(l, EB)
        return 0

    lax.fori_loop(0, n_full, chunk, 0)
    if tail:
        load_chunk(n_full, tail)

    wp = wpr[0]
    for p in range(G // 16):
        idx16 = wp + p * 16 + iota
        pm = idx16 < G
        plsc.store_scatter(st_src, [idx16], jnp.zeros((16,), jnp.int32),
                           mask=pm)
        plsc.store_scatter(st_dstl, [idx16],
                           jnp.full((16,), OWN, jnp.int32), mask=pm)
    _drain(hs, acc_flat, st_src, st_dstl, rows, gsem)

    pltpu.sync_copy(acc_flat.at[pl.ds(0, OWN * H)],
                    out_flat.at[pl.ds(sid * (OWN * H), OWN * H)])


def _prop_scratch():
    return [
        pltpu.VMEM((EB,), jnp.int32),
        pltpu.VMEM((EB,), jnp.int32),
        pltpu.VMEM((ST_CAP,), jnp.int32),
        pltpu.VMEM((ST_CAP,), jnp.int32),
        pltpu.VMEM((G, H), jnp.float32),
        pltpu.VMEM((ACC_ROWS * H,), jnp.float32),
        pltpu.SMEM((1,), jnp.int32),
        pltpu.SemaphoreType.DMA,
    ]


def _sc_prop1(hsA, hsB, src, dst):
    @functools.partial(
        pl.kernel,
        mesh=_sc_mesh(),
        compiler_params=pltpu.CompilerParams(needs_layout_passes=False),
        out_type=(jax.ShapeDtypeStruct((NPAD2 * H,), jnp.float32),
                  jax.ShapeDtypeStruct((NPAD2 * H,), jnp.float32)),
        scratch_types=_prop_scratch(),
    )
    def k(hsA_h, hsB_h, src_h, dst_h, outA, outB, src_b, dst_b, st_src,
          st_dstl, rows, acc_flat, wpr, gsem):
        cid = lax.axis_index("c")
        sid = lax.axis_index("s")

        @pl.when(cid == 0)
        def _():
            _owner_scan(hsA_h, outA, src_h, dst_h, src_b, dst_b, st_src,
                        st_dstl, rows, acc_flat, wpr, gsem, sid, 0, E)

        @pl.when(cid == 1)
        def _():
            _owner_scan(hsB_h, outB, src_h, dst_h, src_b, dst_b, st_src,
                        st_dstl, rows, acc_flat, wpr, gsem, sid, 0, E)

    return k(hsA, hsB, src, dst)


def _sc_prop2(hs2, src, dst):
    epc = E // NC

    @functools.partial(
        pl.kernel,
        mesh=_sc_mesh(),
        compiler_params=pltpu.CompilerParams(needs_layout_passes=False),
        out_type=(jax.ShapeDtypeStruct((NPAD2 * H,), jnp.float32),
                  jax.ShapeDtypeStruct((NPAD2 * H,), jnp.float32)),
        scratch_types=_prop_scratch(),
    )
    def k(hs_h, src_h, dst_h, out0, out1, src_b, dst_b, st_src,
          st_dstl, rows, acc_flat, wpr, gsem):
        cid = lax.axis_index("c")
        sid = lax.axis_index("s")

        @pl.when(cid == 0)
        def _():
            _owner_scan(hs_h, out0, src_h, dst_h, src_b, dst_b, st_src,
                        st_dstl, rows, acc_flat, wpr, gsem, sid, 0, epc)

        @pl.when(cid == 1)
        def _():
            _owner_scan(hs_h, out1, src_h, dst_h, src_b, dst_b, st_src,
                        st_dstl, rows, acc_flat, wpr, gsem, sid, epc, epc)

    return k(hs2, src, dst)


def _sc_edge_dot(x, srcg, dstg):
    e2 = srcg.shape[0] * G
    ngrp = srcg.shape[0]
    base_g = ngrp // NW
    extra = ngrp - base_g * NW

    @functools.partial(
        pl.kernel,
        mesh=_sc_mesh(),
        compiler_params=pltpu.CompilerParams(needs_layout_passes=False,
                                             use_tc_tiling_on_sc=False),
        out_type=jax.ShapeDtypeStruct((e2,), jnp.float32),
        scratch_types=[
            pltpu.VMEM((G,), jnp.int32),
            pltpu.VMEM((G,), jnp.int32),
            pltpu.VMEM((G, O), jnp.float32),
            pltpu.VMEM((G, O), jnp.float32),
            pltpu.VMEM((G,), jnp.float32),
            pltpu.SemaphoreType.DMA,
            pltpu.SemaphoreType.DMA,
        ],
    )
    def k(x_h, srcg_h, dstg_h, res, si, di, xs, xd, out_v, sem0, sem1):
        cid = lax.axis_index("c")
        sid = lax.axis_index("s")
        wid = sid * NC + cid

        zero = jnp.zeros((16,), jnp.float32)

        def do_group(g):
            pltpu.sync_copy(srcg_h.at[g], si)
            pltpu.sync_copy(dstg_h.at[g], di)
            cp0 = pltpu.async_copy(x_h.at[si], xs, sem0)
            cp1 = pltpu.async_copy(x_h.at[di], xd, sem1)
            cp0.wait()
            cp1.wait()

            for z in range(G // 16):
                out_v[pl.ds(z * 16, 16)] = zero

            def edge(e, _):
                p = xs[e, pl.ds(0, 16)] * xd[e, pl.ds(0, 16)]
                for j in range(1, O // 16):
                    p = p + xs[e, pl.ds(j * 16, 16)] * xd[e, pl.ds(j * 16, 16)]
                eidx = jnp.broadcast_to(e, (16,)).astype(jnp.int32)
                plsc.addupdate_scatter(out_v, [eidx], p)
                return 0

            lax.fori_loop(0, G, edge, 0)
            pltpu.sync_copy(out_v, res.at[pl.ds(g * G, G)])

        def body(k_, _):
            do_group(wid + NW * k_)
            return 0

        lax.fori_loop(0, base_g, body, 0)

        @pl.when(wid < extra)
        def _():
            do_group(wid + NW * base_g)

    return k(x, srcg, dstg)


def _tc_dinv(part):
    part3 = part.reshape(NW, NPAD // 128, 128)

    def body(p_ref, o_ref):
        deg = jnp.sum(p_ref[...], axis=0) + 1.0
        o_ref[...] = lax.rsqrt(deg)

    out = pl.pallas_call(
        body,
        out_shape=jax.ShapeDtypeStruct((NPAD // 128, 128), jnp.float32),
    )(part3)
    return out.reshape(NPAD)


def _tc_mm_scale(x, w, dinv2, block_rows=2000):
    m, kdim = x.shape
    n = w.shape[1]

    def body(x_ref, w_ref, d_ref, o_ref):
        o_ref[...] = jnp.dot(x_ref[...], w_ref[...],
                             preferred_element_type=jnp.float32) * d_ref[...]

    return pl.pallas_call(
        body,
        grid=(m // block_rows,),
        in_specs=[
            pl.BlockSpec((block_rows, kdim), lambda i: (i, 0)),
            pl.BlockSpec((kdim, n), lambda i: (0, 0)),
            pl.BlockSpec((block_rows, 1), lambda i: (i, 0)),
        ],
        out_specs=pl.BlockSpec((block_rows, n), lambda i: (i, 0)),
        out_shape=jax.ShapeDtypeStruct((m, n), jnp.float32),
    )(x, w, dinv2)


def _tc_layer2(accA, hsA, accB, hsB, dinv2, b1, W2, block_rows=2000):
    m = accA.shape[0]

    def body(aA, hA, aB, hB, d_ref, b_ref, w_ref, o_ref):
        d = d_ref[...]
        tA = jax.nn.relu(d * (aA[...] + hA[...]) + b_ref[...])
        tB = jax.nn.relu(d * (aB[...] + hB[...]) + b_ref[...])
        oA = jnp.dot(tA, w_ref[...], preferred_element_type=jnp.float32) * d
        oB = jnp.dot(tB, w_ref[...], preferred_element_type=jnp.float32) * d
        o_ref[...] = jnp.concatenate([oA, oB], axis=1)

    return pl.pallas_call(
        body,
        grid=(m // block_rows,),
        in_specs=[
            pl.BlockSpec((block_rows, H), lambda i: (i, 0)),
            pl.BlockSpec((block_rows, H), lambda i: (i, 0)),
            pl.BlockSpec((block_rows, H), lambda i: (i, 0)),
            pl.BlockSpec((block_rows, H), lambda i: (i, 0)),
            pl.BlockSpec((block_rows, 1), lambda i: (i, 0)),
            pl.BlockSpec((1, H), lambda i: (0, 0)),
            pl.BlockSpec((H, O), lambda i: (0, 0)),
        ],
        out_specs=pl.BlockSpec((block_rows, 2 * O), lambda i: (i, 0)),
        out_shape=jax.ShapeDtypeStruct((m, 2 * O), jnp.float32),
    )(accA, hsA, accB, hsB, dinv2, b1.reshape(1, H), W2)


def _tc_final(acc0, acc1, hs2, dinv2, b2, Wm, bm, Wa, ba, block_rows=2000):
    m = hs2.shape[0]

    def body(a0, a1, h_ref, d_ref, b2_ref, wm_ref, bm_ref, wa_ref, ba_ref,
             x_ref, att_ref):
        d = d_ref[...]
        xc = d * (a0[...] + a1[...] + h_ref[...]) + b2_ref[...]
        x_ref[...] = jnp.dot(xc, wm_ref[...],
                             preferred_element_type=jnp.float32) + bm_ref[...]
        x2 = xc[:, O:]
        t = jnp.dot(x2, wa_ref[...],
                    preferred_element_type=jnp.float32) + ba_ref[...]
        tm = jnp.max(t, axis=1, keepdims=True)
        tt = t - tm
        att_ref[...] = tt - jnp.log(jnp.sum(jnp.exp(tt), axis=1,
                                            keepdims=True))

    b22 = jnp.concatenate([b2, b2]).reshape(1, 2 * O)
    return pl.pallas_call(
        body,
        grid=(m // block_rows,),
        in_specs=[
            pl.BlockSpec((block_rows, 2 * O), lambda i: (i, 0)),
            pl.BlockSpec((block_rows, 2 * O), lambda i: (i, 0)),
            pl.BlockSpec((block_rows, 2 * O), lambda i: (i, 0)),
            pl.BlockSpec((block_rows, 1), lambda i: (i, 0)),
            pl.BlockSpec((1, 2 * O), lambda i: (0, 0)),
            pl.BlockSpec((2 * O, O), lambda i: (0, 0)),
            pl.BlockSpec((1, O), lambda i: (0, 0)),
            pl.BlockSpec((O, C), lambda i: (0, 0)),
            pl.BlockSpec((1, C), lambda i: (0, 0)),
        ],
        out_specs=[
            pl.BlockSpec((block_rows, O), lambda i: (i, 0)),
            pl.BlockSpec((block_rows, C), lambda i: (i, 0)),
        ],
        out_shape=[
            jax.ShapeDtypeStruct((m, O), jnp.float32),
            jax.ShapeDtypeStruct((m, C), jnp.float32),
        ],
    )(acc0, acc1, hs2, dinv2, b22, Wm, bm.reshape(1, O), Wa,
      ba.reshape(1, C))


def kernel(x_A, x_B, train_pos_edge_index, pos_edge_index, neg_edge_index,
           W1, b1, W2, b2, Wm, bm, Wa, ba):
    src = train_pos_edge_index[0]
    dst = train_pos_edge_index[1]

    part = _sc_degree(dst)
    dinv = _tc_dinv(part)[:N]
    dinv2 = dinv[:, None]

    hsA = _tc_mm_scale(x_A, W1, dinv2)
    hsB = _tc_mm_scale(x_B, W1, dinv2)

    accA, accB = _sc_prop1(hsA, hsB, src, dst)
    accA = accA.reshape(NPAD2, H)[:N]
    accB = accB.reshape(NPAD2, H)[:N]

    hs2 = _tc_layer2(accA, hsA, accB, hsB, dinv2, b1, W2)

    acc0, acc1 = _sc_prop2(hs2, src, dst)
    acc0 = acc0.reshape(NPAD2, H)[:N]
    acc1 = acc1.reshape(NPAD2, H)[:N]

    x, att = _tc_final(acc0, acc1, hs2, dinv2, b2, Wm, bm, Wa, ba)

    tot = jnp.concatenate([pos_edge_index, neg_edge_index], axis=-1)
    e2 = tot.shape[1]
    res = _sc_edge_dot(x, tot[0].reshape(e2 // G, G), tot[1].reshape(e2 // G, G))
    return res, att


# speedup vs baseline: 1.1410x; 1.1030x over previous
import functools

import jax
import jax.numpy as jnp
from jax import lax
from jax.experimental import pallas as pl
from jax.experimental.pallas import tpu as pltpu
from jax.experimental.pallas import tpu_sc as plsc

N = 10000
NPAD = 10112
RPT = NPAD // 16
E = 160000
G = 128
NGRP = E // G
F_IN = 256
H = 128
O = 64
C = 16

NC = 2
NS = 16
NW = NC * NS


def _sc_mesh():
    return plsc.VectorSubcoreMesh(core_axis_name="c", subcore_axis_name="s")


def _zero_vmem2d(buf, rows, cols):
    zero = jnp.zeros((16,), jnp.float32)

    def zb(i, _):
        r = i // (cols // 16)
        c = i % (cols // 16)
        buf[r, pl.ds(c * 16, 16)] = zero
        return 0

    lax.fori_loop(0, rows * (cols // 16), zb, 0)


def _zero_acc_slice(zbuf, acc_sh, sid):
    nfull = RPT // 64
    for t in range(nfull):
        pltpu.sync_copy(zbuf, acc_sh.at[pl.ds(sid * RPT + t * 64, 64)])
    rem = RPT - nfull * 64
    if rem:
        pltpu.sync_copy(zbuf.at[pl.ds(0, rem)],
                        acc_sh.at[pl.ds(sid * RPT + nfull * 64, rem)])


def _sc_degree(dst):
    ept = E // NW
    full = ept // 16
    tail = ept - full * 16

    @functools.partial(
        pl.kernel,
        mesh=_sc_mesh(),
        compiler_params=pltpu.CompilerParams(needs_layout_passes=False),
        out_type=jax.ShapeDtypeStruct((NW, NPAD), jnp.float32),
        scratch_types=[
            pltpu.VMEM((NPAD,), jnp.float32),
            pltpu.VMEM((ept + 16,), jnp.int32),
        ],
    )
    def k(dst_hbm, out_hbm, hist, idx):
        cid = lax.axis_index("c")
        sid = lax.axis_index("s")
        wid = sid * NC + cid

        zero = jnp.zeros((16,), jnp.float32)

        def zbody(i, _):
            hist[pl.ds(i * 16, 16)] = zero
            return 0

        lax.fori_loop(0, NPAD // 16, zbody, 0)

        pltpu.sync_copy(dst_hbm.at[pl.ds(wid * ept, ept)],
                        idx.at[pl.ds(0, ept)])

        ones = jnp.ones((16,), jnp.float32)

        def hbody(i, _):
            v = idx[pl.ds(i * 16, 16)]
            plsc.addupdate_scatter(hist, [v], ones)
            return 0

        lax.fori_loop(0, full, hbody, 0)

        if tail:
            lanes = lax.iota(jnp.int32, 16)
            tmask = lanes < tail
            v = idx[pl.ds(full * 16, 16)]
            v = jnp.where(tmask, v, 0)
            plsc.addupdate_scatter(hist, [v], ones, mask=tmask)

        pltpu.sync_copy(hist, out_hbm.at[wid])

    return k(dst)


OWN = 640
NPAD2 = 16 * OWN
ACC_ROWS = OWN + 8
ST_CAP = 272
EB = 1024
K_BUF = 2


def _drain(hs, acc_flat, st_src, st_dstl, rows, gsem):
    pltpu.async_copy(hs.at[st_src.at[pl.ds(0, G)]], rows, gsem).wait()
    iota = lax.iota(jnp.int32, 16)

    def edge(e, _):
        base = plsc.load_gather(
            st_dstl, [jnp.broadcast_to(e, (16,)).astype(jnp.int32)])
        addr = base * H + iota
        for j in range(H // 16):
            plsc.addupdate_scatter(acc_flat, [addr + j * 16],
                                   rows[e, pl.ds(j * 16, 16)])
        return 0

    lax.fori_loop(0, G, edge, 0)


def _owner_scan(hs, out_flat, src_h, dst_h, src_b, dst_b, st_src, st_dstl,
                rows, acc_flat, wpr, gsem, sid, eoff, n_edges):
    iota = lax.iota(jnp.int32, 16)
    zero = jnp.zeros((16,), jnp.float32)

    def zb(i, _):
        acc_flat[pl.ds(i * 16, 16)] = zero
        return 0

    lax.fori_loop(0, (ACC_ROWS * H) // 16, zb, 0)
    wpr[0] = 0

    my_base = sid * OWN

    def subchunk(c, wp):
        s16 = src_b[pl.ds(c * 16, 16)]
        d16 = dst_b[pl.ds(c * 16, 16)]
        own16 = jnp.right_shift(d16 * 52429, 25)
        m = own16 == sid
        plsc.store_compressed(st_src.at[pl.ds(wp, 16)], s16, mask=m)
        plsc.store_compressed(st_dstl.at[pl.ds(wp, 16)], d16 - my_base,
                              mask=m)
        return wp + lax.reduce_sum(m.astype(jnp.int32), axes=(0,))

    UNROLL = 8

    def block(i, _):
        wp = wpr[0]
        for u in range(UNROLL):
            wp = subchunk(i * UNROLL + u, wp)

        @pl.when(wp >= G)
        def _():
            _drain(hs, acc_flat, st_src, st_dstl, rows, gsem)
            for t in range(8):
                t0 = st_src[pl.ds(G + t * 16, 16)]
                t1 = st_dstl[pl.ds(G + t * 16, 16)]
                st_src[pl.ds(t * 16, 16)] = t0
                st_dstl[pl.ds(t * 16, 16)] = t1
            wpr[0] = wp - G

        @pl.when(wp < G)
        def _():
            wpr[0] = wp

        return 0

    n_full = n_edges // EB
    tail = n_edges - n_full * EB

    def load_chunk(l, nb):
        pltpu.sync_copy(src_h.at[pl.ds(eoff + l * EB, nb)],
                        src_b.at[pl.ds(0, nb)])
        pltpu.sync_copy(dst_h.at[pl.ds(eoff + l * EB, nb)],
                        dst_b.at[pl.ds(0, nb)])
        lax.fori_loop(0, nb // (16 * UNROLL), block, 0)

    def chunk(l, _):
        load_chunk(l, EB)
        return 0

    lax.fori_loop(0, n_full, chunk, 0)
    if tail:
        load_chunk(n_full, tail)

    wp = wpr[0]
    for p in range(G // 16):
        idx16 = wp + p * 16 + iota
        pm = idx16 < G
        plsc.store_scatter(st_src, [idx16], jnp.zeros((16,), jnp.int32),
                           mask=pm)
        plsc.store_scatter(st_dstl, [idx16],
                           jnp.full((16,), OWN, jnp.int32), mask=pm)
    _drain(hs, acc_flat, st_src, st_dstl, rows, gsem)

    pltpu.sync_copy(acc_flat.at[pl.ds(0, OWN * H)],
                    out_flat.at[pl.ds(sid * (OWN * H), OWN * H)])


def _prop_scratch():
    return [
        pltpu.VMEM((EB,), jnp.int32),
        pltpu.VMEM((EB,), jnp.int32),
        pltpu.VMEM((ST_CAP,), jnp.int32),
        pltpu.VMEM((ST_CAP,), jnp.int32),
        pltpu.VMEM((G, H), jnp.float32),
        pltpu.VMEM((ACC_ROWS * H,), jnp.float32),
        pltpu.SMEM((1,), jnp.int32),
        pltpu.SemaphoreType.DMA,
    ]


def _sc_prop1(hsA, hsB, src, dst):
    @functools.partial(
        pl.kernel,
        mesh=_sc_mesh(),
        compiler_params=pltpu.CompilerParams(needs_layout_passes=False),
        out_type=(jax.ShapeDtypeStruct((NPAD2 * H,), jnp.float32),
                  jax.ShapeDtypeStruct((NPAD2 * H,), jnp.float32)),
        scratch_types=_prop_scratch(),
    )
    def k(hsA_h, hsB_h, src_h, dst_h, outA, outB, src_b, dst_b, st_src,
          st_dstl, rows, acc_flat, wpr, gsem):
        cid = lax.axis_index("c")
        sid = lax.axis_index("s")

        @pl.when(cid == 0)
        def _():
            _owner_scan(hsA_h, outA, src_h, dst_h, src_b, dst_b, st_src,
                        st_dstl, rows, acc_flat, wpr, gsem, sid, 0, E)

        @pl.when(cid == 1)
        def _():
            _owner_scan(hsB_h, outB, src_h, dst_h, src_b, dst_b, st_src,
                        st_dstl, rows, acc_flat, wpr, gsem, sid, 0, E)

    return k(hsA, hsB, src, dst)


def _sc_prop2(hs2, src, dst):
    epc = E // NC

    @functools.partial(
        pl.kernel,
        mesh=_sc_mesh(),
        compiler_params=pltpu.CompilerParams(needs_layout_passes=False),
        out_type=(jax.ShapeDtypeStruct((NPAD2 * H,), jnp.float32),
                  jax.ShapeDtypeStruct((NPAD2 * H,), jnp.float32)),
        scratch_types=_prop_scratch(),
    )
    def k(hs_h, src_h, dst_h, out0, out1, src_b, dst_b, st_src,
          st_dstl, rows, acc_flat, wpr, gsem):
        cid = lax.axis_index("c")
        sid = lax.axis_index("s")

        @pl.when(cid == 0)
        def _():
            _owner_scan(hs_h, out0, src_h, dst_h, src_b, dst_b, st_src,
                        st_dstl, rows, acc_flat, wpr, gsem, sid, 0, epc)

        @pl.when(cid == 1)
        def _():
            _owner_scan(hs_h, out1, src_h, dst_h, src_b, dst_b, st_src,
                        st_dstl, rows, acc_flat, wpr, gsem, sid, epc, epc)

    return k(hs2, src, dst)


def _sc_edge_dot(x, srcg, dstg):
    e2 = srcg.shape[0] * G
    ngrp = srcg.shape[0]
    base_g = ngrp // NW
    extra = ngrp - base_g * NW

    @functools.partial(
        pl.kernel,
        mesh=_sc_mesh(),
        compiler_params=pltpu.CompilerParams(needs_layout_passes=False,
                                             use_tc_tiling_on_sc=False),
        out_type=jax.ShapeDtypeStruct((e2,), jnp.float32),
        scratch_types=[
            pltpu.VMEM((G,), jnp.int32),
            pltpu.VMEM((G,), jnp.int32),
            pltpu.VMEM((G, O), jnp.float32),
            pltpu.VMEM((G, O), jnp.float32),
            pltpu.VMEM((G,), jnp.float32),
            pltpu.SemaphoreType.DMA,
            pltpu.SemaphoreType.DMA,
        ],
    )
    def k(x_h, srcg_h, dstg_h, res, si, di, xs, xd, out_v, sem0, sem1):
        cid = lax.axis_index("c")
        sid = lax.axis_index("s")
        wid = sid * NC + cid

        zero = jnp.zeros((16,), jnp.float32)

        def do_group(g):
            pltpu.sync_copy(srcg_h.at[g], si)
            pltpu.sync_copy(dstg_h.at[g], di)
            cp0 = pltpu.async_copy(x_h.at[si], xs, sem0)
            cp1 = pltpu.async_copy(x_h.at[di], xd, sem1)
            cp0.wait()
            cp1.wait()

            for z in range(G // 16):
                out_v[pl.ds(z * 16, 16)] = zero

            def edge(e, _):
                p = xs[e, pl.ds(0, 16)] * xd[e, pl.ds(0, 16)]
                for j in range(1, O // 16):
                    p = p + xs[e, pl.ds(j * 16, 16)] * xd[e, pl.ds(j * 16, 16)]
                eidx = jnp.broadcast_to(e, (16,)).astype(jnp.int32)
                plsc.addupdate_scatter(out_v, [eidx], p)
                return 0

            lax.fori_loop(0, G, edge, 0)
            pltpu.sync_copy(out_v, res.at[pl.ds(g * G, G)])

        def body(k_, _):
            do_group(wid + NW * k_)
            return 0

        lax.fori_loop(0, base_g, body, 0)

        @pl.when(wid < extra)
        def _():
            do_group(wid + NW * base_g)

    return k(x, srcg, dstg)


def _tc_dinv(part):
    part3 = part.reshape(NW, NPAD // 128, 128)

    def body(p_ref, o_ref):
        deg = jnp.sum(p_ref[...], axis=0) + 1.0
        o_ref[...] = lax.rsqrt(deg)

    out = pl.pallas_call(
        body,
        out_shape=jax.ShapeDtypeStruct((NPAD // 128, 128), jnp.float32),
    )(part3)
    return out.reshape(NPAD)


def _tc_mm_scale(x, w, dinv2, block_rows=2000):
    m, kdim = x.shape
    n = w.shape[1]

    def body(x_ref, w_ref, d_ref, o_ref):
        o_ref[...] = jnp.dot(x_ref[...], w_ref[...],
                             preferred_element_type=jnp.float32) * d_ref[...]

    return pl.pallas_call(
        body,
        grid=(m // block_rows,),
        in_specs=[
            pl.BlockSpec((block_rows, kdim), lambda i: (i, 0)),
            pl.BlockSpec((kdim, n), lambda i: (0, 0)),
            pl.BlockSpec((block_rows, 1), lambda i: (i, 0)),
        ],
        out_specs=pl.BlockSpec((block_rows, n), lambda i: (i, 0)),
        out_shape=jax.ShapeDtypeStruct((m, n), jnp.float32),
    )(x, w, dinv2)


def _tc_layer2(accA, hsA, accB, hsB, dinv2, b1, W2, block_rows=2000):
    m = accA.shape[0]

    def body(aA, hA, aB, hB, d_ref, b_ref, w_ref, o_ref):
        d = d_ref[...]
        tA = jax.nn.relu(d * (aA[...] + hA[...]) + b_ref[...])
        tB = jax.nn.relu(d * (aB[...] + hB[...]) + b_ref[...])
        oA = jnp.dot(tA, w_ref[...], preferred_element_type=jnp.float32) * d
        oB = jnp.dot(tB, w_ref[...], preferred_element_type=jnp.float32) * d
        o_ref[...] = jnp.concatenate([oA, oB], axis=1)

    return pl.pallas_call(
        body,
        grid=(m // block_rows,),
        in_specs=[
            pl.BlockSpec((block_rows, H), lambda i: (i, 0)),
            pl.BlockSpec((block_rows, H), lambda i: (i, 0)),
            pl.BlockSpec((block_rows, H), lambda i: (i, 0)),
            pl.BlockSpec((block_rows, H), lambda i: (i, 0)),
            pl.BlockSpec((block_rows, 1), lambda i: (i, 0)),
            pl.BlockSpec((1, H), lambda i: (0, 0)),
            pl.BlockSpec((H, O), lambda i: (0, 0)),
        ],
        out_specs=pl.BlockSpec((block_rows, 2 * O), lambda i: (i, 0)),
        out_shape=jax.ShapeDtypeStruct((m, 2 * O), jnp.float32),
    )(accA, hsA, accB, hsB, dinv2, b1.reshape(1, H), W2)


def _tc_final(acc0, acc1, hs2, dinv2, b2, Wm, bm, Wa, ba, block_rows=2000):
    m = hs2.shape[0]

    def body(a0, a1, h_ref, d_ref, b2_ref, wm_ref, bm_ref, wa_ref, ba_ref,
             x_ref, att_ref):
        d = d_ref[...]
        xc = d * (a0[...] + a1[...] + h_ref[...]) + b2_ref[...]
        x_ref[...] = jnp.dot(xc, wm_ref[...],
                             preferred_element_type=jnp.float32) + bm_ref[...]
        x2 = xc[:, O:]
        t = jnp.dot(x2, wa_ref[...],
                    preferred_element_type=jnp.float32) + ba_ref[...]
        tm = jnp.max(t, axis=1, keepdims=True)
        tt = t - tm
        att_ref[...] = tt - jnp.log(jnp.sum(jnp.exp(tt), axis=1,
                                            keepdims=True))

    b22 = jnp.concatenate([b2, b2]).reshape(1, 2 * O)
    return pl.pallas_call(
        body,
        grid=(m // block_rows,),
        in_specs=[
            pl.BlockSpec((block_rows, 2 * O), lambda i: (i, 0)),
            pl.BlockSpec((block_rows, 2 * O), lambda i: (i, 0)),
            pl.BlockSpec((block_rows, 2 * O), lambda i: (i, 0)),
            pl.BlockSpec((block_rows, 1), lambda i: (i, 0)),
            pl.BlockSpec((1, 2 * O), lambda i: (0, 0)),
            pl.BlockSpec((2 * O, O), lambda i: (0, 0)),
            pl.BlockSpec((1, O), lambda i: (0, 0)),
            pl.BlockSpec((O, C), lambda i: (0, 0)),
            pl.BlockSpec((1, C), lambda i: (0, 0)),
        ],
        out_specs=[
            pl.BlockSpec((block_rows, O), lambda i: (i, 0)),
            pl.BlockSpec((block_rows, C), lambda i: (i, 0)),
        ],
        out_shape=[
            jax.ShapeDtypeStruct((m, O), jnp.float32),
            jax.ShapeDtypeStruct((m, C), jnp.float32),
        ],
    )(acc0, acc1, hs2, dinv2, b22, Wm, bm.reshape(1, O), Wa,
      ba.reshape(1, C))


def kernel(x_A, x_B, train_pos_edge_index, pos_edge_index, neg_edge_index,
           W1, b1, W2, b2, Wm, bm, Wa, ba):
    src = train_pos_edge_index[0]
    dst = train_pos_edge_index[1]

    part = _sc_degree(dst)
    dinv = _tc_dinv(part)[:N]
    dinv2 = dinv[:, None]

    hsA = _tc_mm_scale(x_A, W1, dinv2)
    hsB = _tc_mm_scale(x_B, W1, dinv2)

    accA, accB = _sc_prop1(hsA, hsB, src, dst)
    accA = accA.reshape(NPAD2, H)[:N]
    accB = accB.reshape(NPAD2, H)[:N]

    hs2 = _tc_layer2(accA, hsA, accB, hsB, dinv2, b1, W2)

    acc0, acc1 = _sc_prop2(hs2, src, dst)
    acc0 = acc0.reshape(NPAD2, H)[:N]
    acc1 = acc1.reshape(NPAD2, H)[:N]

    x, att = _tc_final(acc0, acc1, hs2, dinv2, b2, Wm, bm, Wa, ba)

    tot = jnp.concatenate([pos_edge_index, neg_edge_index], axis=-1)
    e2 = tot.shape[1]
    res = _sc_edge_dot(x, tot[0].reshape(e2 // G, G), tot[1].reshape(e2 // G, G))
    return res, att


# speedup vs baseline: 1.2441x; 1.0903x over previous
import functools

import jax
import jax.numpy as jnp
from jax import lax
from jax.experimental import pallas as pl
from jax.experimental.pallas import tpu as pltpu
from jax.experimental.pallas import tpu_sc as plsc

N = 10000
NPAD = 10112
RPT = NPAD // 16
E = 160000
G = 128
NGRP = E // G
F_IN = 256
H = 128
O = 64
C = 16

NC = 2
NS = 16
NW = NC * NS


def _sc_mesh():
    return plsc.VectorSubcoreMesh(core_axis_name="c", subcore_axis_name="s")


def _zero_vmem2d(buf, rows, cols):
    zero = jnp.zeros((16,), jnp.float32)

    def zb(i, _):
        r = i // (cols // 16)
        c = i % (cols // 16)
        buf[r, pl.ds(c * 16, 16)] = zero
        return 0

    lax.fori_loop(0, rows * (cols // 16), zb, 0)


def _zero_acc_slice(zbuf, acc_sh, sid):
    nfull = RPT // 64
    for t in range(nfull):
        pltpu.sync_copy(zbuf, acc_sh.at[pl.ds(sid * RPT + t * 64, 64)])
    rem = RPT - nfull * 64
    if rem:
        pltpu.sync_copy(zbuf.at[pl.ds(0, rem)],
                        acc_sh.at[pl.ds(sid * RPT + nfull * 64, rem)])


def _sc_degree(dst):
    ept = E // NW
    full = ept // 16
    tail = ept - full * 16

    @functools.partial(
        pl.kernel,
        mesh=_sc_mesh(),
        compiler_params=pltpu.CompilerParams(needs_layout_passes=False),
        out_type=jax.ShapeDtypeStruct((NW, NPAD), jnp.float32),
        scratch_types=[
            pltpu.VMEM((NPAD,), jnp.float32),
            pltpu.VMEM((ept + 16,), jnp.int32),
        ],
    )
    def k(dst_hbm, out_hbm, hist, idx):
        cid = lax.axis_index("c")
        sid = lax.axis_index("s")
        wid = sid * NC + cid

        zero = jnp.zeros((16,), jnp.float32)

        def zbody(i, _):
            hist[pl.ds(i * 16, 16)] = zero
            return 0

        lax.fori_loop(0, NPAD // 16, zbody, 0)

        pltpu.sync_copy(dst_hbm.at[pl.ds(wid * ept, ept)],
                        idx.at[pl.ds(0, ept)])

        ones = jnp.ones((16,), jnp.float32)

        def hbody(i, _):
            v = idx[pl.ds(i * 16, 16)]
            plsc.addupdate_scatter(hist, [v], ones)
            return 0

        lax.fori_loop(0, full, hbody, 0)

        if tail:
            lanes = lax.iota(jnp.int32, 16)
            tmask = lanes < tail
            v = idx[pl.ds(full * 16, 16)]
            v = jnp.where(tmask, v, 0)
            plsc.addupdate_scatter(hist, [v], ones, mask=tmask)

        pltpu.sync_copy(hist, out_hbm.at[wid])

    return k(dst)


OWN = 640
NPAD2 = 16 * OWN
ACC_ROWS = OWN + 8
ST_CAP = 272
EB = 1024
K_BUF = 2


def _drain(hs, acc_flat, st_src, st_dstl, rows, gsem):
    pltpu.async_copy(hs.at[st_src.at[pl.ds(0, G)]], rows, gsem).wait()
    iota = lax.iota(jnp.int32, 16)

    def edge(e, _):
        base = plsc.load_gather(
            st_dstl, [jnp.broadcast_to(e, (16,)).astype(jnp.int32)])
        addr = base * H + iota
        for j in range(H // 16):
            plsc.addupdate_scatter(acc_flat, [addr + j * 16],
                                   rows[e, pl.ds(j * 16, 16)])
        return 0

    lax.fori_loop(0, G, edge, 0)


def _owner_scan(hs, out_flat, src_h, dst_h, src_b, dst_b, st_src, st_dstl,
                rows, acc_flat, wpr, gsem, sid, eoff, n_edges):
    iota = lax.iota(jnp.int32, 16)
    zero = jnp.zeros((16,), jnp.float32)

    def zb(i, _):
        acc_flat[pl.ds(i * 16, 16)] = zero
        return 0

    lax.fori_loop(0, (ACC_ROWS * H) // 16, zb, 0)
    wpr[0] = 0

    my_base = sid * OWN

    UNROLL = 8

    def block(i, _):
        wp = wpr[0]
        vals = []
        for u in range(UNROLL):
            c = i * UNROLL + u
            s16 = src_b[pl.ds(c * 16, 16)]
            d16 = dst_b[pl.ds(c * 16, 16)]
            m = jnp.right_shift(d16 * 52429, 25) == sid
            cnt = lax.reduce_sum(m.astype(jnp.int32), axes=(0,))
            vals.append((s16, d16, m, cnt))
        for s16, d16, m, cnt in vals:
            plsc.store_compressed(st_src.at[pl.ds(wp, 16)], s16, mask=m)
            plsc.store_compressed(st_dstl.at[pl.ds(wp, 16)], d16 - my_base,
                                  mask=m)
            wp = wp + cnt

        @pl.when(wp >= G)
        def _():
            _drain(hs, acc_flat, st_src, st_dstl, rows, gsem)
            for t in range(8):
                t0 = st_src[pl.ds(G + t * 16, 16)]
                t1 = st_dstl[pl.ds(G + t * 16, 16)]
                st_src[pl.ds(t * 16, 16)] = t0
                st_dstl[pl.ds(t * 16, 16)] = t1
            wpr[0] = wp - G

        @pl.when(wp < G)
        def _():
            wpr[0] = wp

        return 0

    n_full = n_edges // EB
    tail = n_edges - n_full * EB

    def load_chunk(l, nb):
        pltpu.sync_copy(src_h.at[pl.ds(eoff + l * EB, nb)],
                        src_b.at[pl.ds(0, nb)])
        pltpu.sync_copy(dst_h.at[pl.ds(eoff + l * EB, nb)],
                        dst_b.at[pl.ds(0, nb)])
        lax.fori_loop(0, nb // (16 * UNROLL), block, 0)

    def chunk(l, _):
        load_chunk(l, EB)
        return 0

    lax.fori_loop(0, n_full, chunk, 0)
    if tail:
        load_chunk(n_full, tail)

    wp = wpr[0]
    for p in range(G // 16):
        idx16 = wp + p * 16 + iota
        pm = idx16 < G
        plsc.store_scatter(st_src, [idx16], jnp.zeros((16,), jnp.int32),
                           mask=pm)
        plsc.store_scatter(st_dstl, [idx16],
                           jnp.full((16,), OWN, jnp.int32), mask=pm)
    _drain(hs, acc_flat, st_src, st_dstl, rows, gsem)

    pltpu.sync_copy(acc_flat.at[pl.ds(0, OWN * H)],
                    out_flat.at[pl.ds(sid * (OWN * H), OWN * H)])


def _prop_scratch():
    return [
        pltpu.VMEM((EB,), jnp.int32),
        pltpu.VMEM((EB,), jnp.int32),
        pltpu.VMEM((ST_CAP,), jnp.int32),
        pltpu.VMEM((ST_CAP,), jnp.int32),
        pltpu.VMEM((G, H), jnp.float32),
        pltpu.VMEM((ACC_ROWS * H,), jnp.float32),
        pltpu.SMEM((1,), jnp.int32),
        pltpu.SemaphoreType.DMA,
    ]


def _sc_prop1(hsA, hsB, src, dst):
    @functools.partial(
        pl.kernel,
        mesh=_sc_mesh(),
        compiler_params=pltpu.CompilerParams(needs_layout_passes=False),
        out_type=(jax.ShapeDtypeStruct((NPAD2 * H,), jnp.float32),
                  jax.ShapeDtypeStruct((NPAD2 * H,), jnp.float32)),
        scratch_types=_prop_scratch(),
    )
    def k(hsA_h, hsB_h, src_h, dst_h, outA, outB, src_b, dst_b, st_src,
          st_dstl, rows, acc_flat, wpr, gsem):
        cid = lax.axis_index("c")
        sid = lax.axis_index("s")

        @pl.when(cid == 0)
        def _():
            _owner_scan(hsA_h, outA, src_h, dst_h, src_b, dst_b, st_src,
                        st_dstl, rows, acc_flat, wpr, gsem, sid, 0, E)

        @pl.when(cid == 1)
        def _():
            _owner_scan(hsB_h, outB, src_h, dst_h, src_b, dst_b, st_src,
                        st_dstl, rows, acc_flat, wpr, gsem, sid, 0, E)

    return k(hsA, hsB, src, dst)


def _sc_prop2(hs2, src, dst):
    epc = E // NC

    @functools.partial(
        pl.kernel,
        mesh=_sc_mesh(),
        compiler_params=pltpu.CompilerParams(needs_layout_passes=False),
        out_type=(jax.ShapeDtypeStruct((NPAD2 * H,), jnp.float32),
                  jax.ShapeDtypeStruct((NPAD2 * H,), jnp.float32)),
        scratch_types=_prop_scratch(),
    )
    def k(hs_h, src_h, dst_h, out0, out1, src_b, dst_b, st_src,
          st_dstl, rows, acc_flat, wpr, gsem):
        cid = lax.axis_index("c")
        sid = lax.axis_index("s")

        @pl.when(cid == 0)
        def _():
            _owner_scan(hs_h, out0, src_h, dst_h, src_b, dst_b, st_src,
                        st_dstl, rows, acc_flat, wpr, gsem, sid, 0, epc)

        @pl.when(cid == 1)
        def _():
            _owner_scan(hs_h, out1, src_h, dst_h, src_b, dst_b, st_src,
                        st_dstl, rows, acc_flat, wpr, gsem, sid, epc, epc)

    return k(hs2, src, dst)


def _sc_edge_dot(x, srcg, dstg):
    e2 = srcg.shape[0] * G
    ngrp = srcg.shape[0]
    base_g = ngrp // NW
    extra = ngrp - base_g * NW

    @functools.partial(
        pl.kernel,
        mesh=_sc_mesh(),
        compiler_params=pltpu.CompilerParams(needs_layout_passes=False,
                                             use_tc_tiling_on_sc=False),
        out_type=jax.ShapeDtypeStruct((e2,), jnp.float32),
        scratch_types=[
            pltpu.VMEM((G,), jnp.int32),
            pltpu.VMEM((G,), jnp.int32),
            pltpu.VMEM((G, O), jnp.float32),
            pltpu.VMEM((G, O), jnp.float32),
            pltpu.VMEM((G,), jnp.float32),
            pltpu.SemaphoreType.DMA,
            pltpu.SemaphoreType.DMA,
        ],
    )
    def k(x_h, srcg_h, dstg_h, res, si, di, xs, xd, out_v, sem0, sem1):
        cid = lax.axis_index("c")
        sid = lax.axis_index("s")
        wid = sid * NC + cid

        zero = jnp.zeros((16,), jnp.float32)

        def do_group(g):
            pltpu.sync_copy(srcg_h.at[g], si)
            pltpu.sync_copy(dstg_h.at[g], di)
            cp0 = pltpu.async_copy(x_h.at[si], xs, sem0)
            cp1 = pltpu.async_copy(x_h.at[di], xd, sem1)
            cp0.wait()
            cp1.wait()

            for z in range(G // 16):
                out_v[pl.ds(z * 16, 16)] = zero

            def edge(e, _):
                p = xs[e, pl.ds(0, 16)] * xd[e, pl.ds(0, 16)]
                for j in range(1, O // 16):
                    p = p + xs[e, pl.ds(j * 16, 16)] * xd[e, pl.ds(j * 16, 16)]
                eidx = jnp.broadcast_to(e, (16,)).astype(jnp.int32)
                plsc.addupdate_scatter(out_v, [eidx], p)
                return 0

            lax.fori_loop(0, G, edge, 0)
            pltpu.sync_copy(out_v, res.at[pl.ds(g * G, G)])

        def body(k_, _):
            do_group(wid + NW * k_)
            return 0

        lax.fori_loop(0, base_g, body, 0)

        @pl.when(wid < extra)
        def _():
            do_group(wid + NW * base_g)

    return k(x, srcg, dstg)


def _tc_dinv(part):
    part3 = part.reshape(NW, NPAD // 128, 128)

    def body(p_ref, o_ref):
        deg = jnp.sum(p_ref[...], axis=0) + 1.0
        o_ref[...] = lax.rsqrt(deg)

    out = pl.pallas_call(
        body,
        out_shape=jax.ShapeDtypeStruct((NPAD // 128, 128), jnp.float32),
    )(part3)
    return out.reshape(NPAD)


def _tc_mm_scale(x, w, dinv2, block_rows=2000):
    m, kdim = x.shape
    n = w.shape[1]

    def body(x_ref, w_ref, d_ref, o_ref):
        o_ref[...] = jnp.dot(x_ref[...], w_ref[...],
                             preferred_element_type=jnp.float32) * d_ref[...]

    return pl.pallas_call(
        body,
        grid=(m // block_rows,),
        in_specs=[
            pl.BlockSpec((block_rows, kdim), lambda i: (i, 0)),
            pl.BlockSpec((kdim, n), lambda i: (0, 0)),
            pl.BlockSpec((block_rows, 1), lambda i: (i, 0)),
        ],
        out_specs=pl.BlockSpec((block_rows, n), lambda i: (i, 0)),
        out_shape=jax.ShapeDtypeStruct((m, n), jnp.float32),
    )(x, w, dinv2)


def _tc_layer2(accA, hsA, accB, hsB, dinv2, b1, W2, block_rows=2000):
    m = accA.shape[0]

    def body(aA, hA, aB, hB, d_ref, b_ref, w_ref, o_ref):
        d = d_ref[...]
        tA = jax.nn.relu(d * (aA[...] + hA[...]) + b_ref[...])
        tB = jax.nn.relu(d * (aB[...] + hB[...]) + b_ref[...])
        oA = jnp.dot(tA, w_ref[...], preferred_element_type=jnp.float32) * d
        oB = jnp.dot(tB, w_ref[...], preferred_element_type=jnp.float32) * d
        o_ref[...] = jnp.concatenate([oA, oB], axis=1)

    return pl.pallas_call(
        body,
        grid=(m // block_rows,),
        in_specs=[
            pl.BlockSpec((block_rows, H), lambda i: (i, 0)),
            pl.BlockSpec((block_rows, H), lambda i: (i, 0)),
            pl.BlockSpec((block_rows, H), lambda i: (i, 0)),
            pl.BlockSpec((block_rows, H), lambda i: (i, 0)),
            pl.BlockSpec((block_rows, 1), lambda i: (i, 0)),
            pl.BlockSpec((1, H), lambda i: (0, 0)),
            pl.BlockSpec((H, O), lambda i: (0, 0)),
        ],
        out_specs=pl.BlockSpec((block_rows, 2 * O), lambda i: (i, 0)),
        out_shape=jax.ShapeDtypeStruct((m, 2 * O), jnp.float32),
    )(accA, hsA, accB, hsB, dinv2, b1.reshape(1, H), W2)


def _tc_final(acc0, acc1, hs2, dinv2, b2, Wm, bm, Wa, ba, block_rows=2000):
    m = hs2.shape[0]

    def body(a0, a1, h_ref, d_ref, b2_ref, wm_ref, bm_ref, wa_ref, ba_ref,
             x_ref, att_ref):
        d = d_ref[...]
        xc = d * (a0[...] + a1[...] + h_ref[...]) + b2_ref[...]
        x_ref[...] = jnp.dot(xc, wm_ref[...],
                             preferred_element_type=jnp.float32) + bm_ref[...]
        x2 = xc[:, O:]
        t = jnp.dot(x2, wa_ref[...],
                    preferred_element_type=jnp.float32) + ba_ref[...]
        tm = jnp.max(t, axis=1, keepdims=True)
        tt = t - tm
        att_ref[...] = tt - jnp.log(jnp.sum(jnp.exp(tt), axis=1,
                                            keepdims=True))

    b22 = jnp.concatenate([b2, b2]).reshape(1, 2 * O)
    return pl.pallas_call(
        body,
        grid=(m // block_rows,),
        in_specs=[
            pl.BlockSpec((block_rows, 2 * O), lambda i: (i, 0)),
            pl.BlockSpec((block_rows, 2 * O), lambda i: (i, 0)),
            pl.BlockSpec((block_rows, 2 * O), lambda i: (i, 0)),
            pl.BlockSpec((block_rows, 1), lambda i: (i, 0)),
            pl.BlockSpec((1, 2 * O), lambda i: (0, 0)),
            pl.BlockSpec((2 * O, O), lambda i: (0, 0)),
            pl.BlockSpec((1, O), lambda i: (0, 0)),
            pl.BlockSpec((O, C), lambda i: (0, 0)),
            pl.BlockSpec((1, C), lambda i: (0, 0)),
        ],
        out_specs=[
            pl.BlockSpec((block_rows, O), lambda i: (i, 0)),
            pl.BlockSpec((block_rows, C), lambda i: (i, 0)),
        ],
        out_shape=[
            jax.ShapeDtypeStruct((m, O), jnp.float32),
            jax.ShapeDtypeStruct((m, C), jnp.float32),
        ],
    )(acc0, acc1, hs2, dinv2, b22, Wm, bm.reshape(1, O), Wa,
      ba.reshape(1, C))


def kernel(x_A, x_B, train_pos_edge_index, pos_edge_index, neg_edge_index,
           W1, b1, W2, b2, Wm, bm, Wa, ba):
    src = train_pos_edge_index[0]
    dst = train_pos_edge_index[1]

    part = _sc_degree(dst)
    dinv = _tc_dinv(part)[:N]
    dinv2 = dinv[:, None]

    hsA = _tc_mm_scale(x_A, W1, dinv2)
    hsB = _tc_mm_scale(x_B, W1, dinv2)

    accA, accB = _sc_prop1(hsA, hsB, src, dst)
    accA = accA.reshape(NPAD2, H)[:N]
    accB = accB.reshape(NPAD2, H)[:N]

    hs2 = _tc_layer2(accA, hsA, accB, hsB, dinv2, b1, W2)

    acc0, acc1 = _sc_prop2(hs2, src, dst)
    acc0 = acc0.reshape(NPAD2, H)[:N]
    acc1 = acc1.reshape(NPAD2, H)[:N]

    x, att = _tc_final(acc0, acc1, hs2, dinv2, b2, Wm, bm, Wa, ba)

    tot = jnp.concatenate([pos_edge_index, neg_edge_index], axis=-1)
    e2 = tot.shape[1]
    res = _sc_edge_dot(x, tot[0].reshape(e2 // G, G), tot[1].reshape(e2 // G, G))
    return res, att


# speedup vs baseline: 1.2784x; 1.0276x over previous
import functools

import jax
import jax.numpy as jnp
from jax import lax
from jax.experimental import pallas as pl
from jax.experimental.pallas import tpu as pltpu
from jax.experimental.pallas import tpu_sc as plsc

N = 10000
NPAD = 10112
RPT = NPAD // 16
E = 160000
G = 128
NGRP = E // G
F_IN = 256
H = 128
O = 64
C = 16

NC = 2
NS = 16
NW = NC * NS


def _sc_mesh():
    return plsc.VectorSubcoreMesh(core_axis_name="c", subcore_axis_name="s")


def _zero_vmem2d(buf, rows, cols):
    zero = jnp.zeros((16,), jnp.float32)

    def zb(i, _):
        r = i // (cols // 16)
        c = i % (cols // 16)
        buf[r, pl.ds(c * 16, 16)] = zero
        return 0

    lax.fori_loop(0, rows * (cols // 16), zb, 0)


def _zero_acc_slice(zbuf, acc_sh, sid):
    nfull = RPT // 64
    for t in range(nfull):
        pltpu.sync_copy(zbuf, acc_sh.at[pl.ds(sid * RPT + t * 64, 64)])
    rem = RPT - nfull * 64
    if rem:
        pltpu.sync_copy(zbuf.at[pl.ds(0, rem)],
                        acc_sh.at[pl.ds(sid * RPT + nfull * 64, rem)])


def _sc_degree(dst):
    ept = E // NW
    full = ept // 16
    tail = ept - full * 16

    @functools.partial(
        pl.kernel,
        mesh=_sc_mesh(),
        compiler_params=pltpu.CompilerParams(needs_layout_passes=False),
        out_type=jax.ShapeDtypeStruct((NW, NPAD), jnp.float32),
        scratch_types=[
            pltpu.VMEM((NPAD,), jnp.float32),
            pltpu.VMEM((ept + 16,), jnp.int32),
        ],
    )
    def k(dst_hbm, out_hbm, hist, idx):
        cid = lax.axis_index("c")
        sid = lax.axis_index("s")
        wid = sid * NC + cid

        zero = jnp.zeros((16,), jnp.float32)

        def zbody(i, _):
            hist[pl.ds(i * 16, 16)] = zero
            return 0

        lax.fori_loop(0, NPAD // 16, zbody, 0)

        pltpu.sync_copy(dst_hbm.at[pl.ds(wid * ept, ept)],
                        idx.at[pl.ds(0, ept)])

        ones = jnp.ones((16,), jnp.float32)

        def hbody(i, _):
            v = idx[pl.ds(i * 16, 16)]
            plsc.addupdate_scatter(hist, [v], ones)
            return 0

        lax.fori_loop(0, full, hbody, 0)

        if tail:
            lanes = lax.iota(jnp.int32, 16)
            tmask = lanes < tail
            v = idx[pl.ds(full * 16, 16)]
            v = jnp.where(tmask, v, 0)
            plsc.addupdate_scatter(hist, [v], ones, mask=tmask)

        pltpu.sync_copy(hist, out_hbm.at[wid])

    return k(dst)


OWN = 640
NPAD2 = 16 * OWN
ACC_ROWS = OWN + 8
ST_CAP = 272
EB = 1024
K_BUF = 2


def _drain(hs, acc_flat, st_src, st_dstl, rows, gsem):
    pltpu.async_copy(hs.at[st_src.at[pl.ds(0, G)]], rows, gsem).wait()
    iota = lax.iota(jnp.int32, 16)

    def edge(e, _):
        base = plsc.load_gather(
            st_dstl, [jnp.broadcast_to(e, (16,)).astype(jnp.int32)])
        addr = base * H + iota
        for j in range(H // 16):
            plsc.addupdate_scatter(acc_flat, [addr + j * 16],
                                   rows[e, pl.ds(j * 16, 16)])
        return 0

    lax.fori_loop(0, G, edge, 0)


def _owner_scan(hs, out_flat, src_h, dst_h, src_b, dst_b, st_src, st_dstl,
                rows, acc_flat, wpr, gsem, sid, eoff, n_edges):
    iota = lax.iota(jnp.int32, 16)
    zero = jnp.zeros((16,), jnp.float32)

    def zb(i, _):
        acc_flat[pl.ds(i * 16, 16)] = zero
        return 0

    lax.fori_loop(0, (ACC_ROWS * H) // 16, zb, 0)
    wpr[0] = 0

    my_base = sid * OWN

    UNROLL = 8

    def block(i, _):
        wp = wpr[0]
        vals = []
        for u in range(UNROLL):
            c = i * UNROLL + u
            s16 = src_b[pl.ds(c * 16, 16)]
            d16 = dst_b[pl.ds(c * 16, 16)]
            m = jnp.right_shift(d16 * 52429, 25) == sid
            cnt = lax.reduce_sum(m.astype(jnp.int32), axes=(0,))
            vals.append((s16, d16, m, cnt))
        for s16, d16, m, cnt in vals:
            plsc.store_compressed(st_src.at[pl.ds(wp, 16)], s16, mask=m)
            plsc.store_compressed(st_dstl.at[pl.ds(wp, 16)], d16 - my_base,
                                  mask=m)
            wp = wp + cnt

        @pl.when(wp >= G)
        def _():
            _drain(hs, acc_flat, st_src, st_dstl, rows, gsem)
            for t in range(8):
                t0 = st_src[pl.ds(G + t * 16, 16)]
                t1 = st_dstl[pl.ds(G + t * 16, 16)]
                st_src[pl.ds(t * 16, 16)] = t0
                st_dstl[pl.ds(t * 16, 16)] = t1
            wpr[0] = wp - G

        @pl.when(wp < G)
        def _():
            wpr[0] = wp

        return 0

    n_full = n_edges // EB
    tail = n_edges - n_full * EB

    def load_chunk(l, nb):
        pltpu.sync_copy(src_h.at[pl.ds(eoff + l * EB, nb)],
                        src_b.at[pl.ds(0, nb)])
        pltpu.sync_copy(dst_h.at[pl.ds(eoff + l * EB, nb)],
                        dst_b.at[pl.ds(0, nb)])
        lax.fori_loop(0, nb // (16 * UNROLL), block, 0)

    def chunk(l, _):
        load_chunk(l, EB)
        return 0

    lax.fori_loop(0, n_full, chunk, 0)
    if tail:
        load_chunk(n_full, tail)

    wp = wpr[0]
    for p in range(G // 16):
        idx16 = wp + p * 16 + iota
        pm = idx16 < G
        plsc.store_scatter(st_src, [idx16], jnp.zeros((16,), jnp.int32),
                           mask=pm)
        plsc.store_scatter(st_dstl, [idx16],
                           jnp.full((16,), OWN, jnp.int32), mask=pm)
    _drain(hs, acc_flat, st_src, st_dstl, rows, gsem)

    pltpu.sync_copy(acc_flat.at[pl.ds(0, OWN * H)],
                    out_flat.at[pl.ds(sid * (OWN * H), OWN * H)])


def _prop_scratch():
    return [
        pltpu.VMEM((EB,), jnp.int32),
        pltpu.VMEM((EB,), jnp.int32),
        pltpu.VMEM((ST_CAP,), jnp.int32),
        pltpu.VMEM((ST_CAP,), jnp.int32),
        pltpu.VMEM((G, H), jnp.float32),
        pltpu.VMEM((ACC_ROWS * H,), jnp.float32),
        pltpu.SMEM((1,), jnp.int32),
        pltpu.SemaphoreType.DMA,
    ]


def _sc_prop1(hsA, hsB, src, dst):
    @functools.partial(
        pl.kernel,
        mesh=_sc_mesh(),
        compiler_params=pltpu.CompilerParams(needs_layout_passes=False),
        out_type=(jax.ShapeDtypeStruct((NPAD2 * H,), jnp.float32),
                  jax.ShapeDtypeStruct((NPAD2 * H,), jnp.float32)),
        scratch_types=_prop_scratch(),
    )
    def k(hsA_h, hsB_h, src_h, dst_h, outA, outB, src_b, dst_b, st_src,
          st_dstl, rows, acc_flat, wpr, gsem):
        cid = lax.axis_index("c")
        sid = lax.axis_index("s")

        @pl.when(cid == 0)
        def _():
            _owner_scan(hsA_h, outA, src_h, dst_h, src_b, dst_b, st_src,
                        st_dstl, rows, acc_flat, wpr, gsem, sid, 0, E)

        @pl.when(cid == 1)
        def _():
            _owner_scan(hsB_h, outB, src_h, dst_h, src_b, dst_b, st_src,
                        st_dstl, rows, acc_flat, wpr, gsem, sid, 0, E)

    return k(hsA, hsB, src, dst)


def _sc_prop2(hs2, src, dst):
    epc = E // NC

    @functools.partial(
        pl.kernel,
        mesh=_sc_mesh(),
        compiler_params=pltpu.CompilerParams(needs_layout_passes=False),
        out_type=(jax.ShapeDtypeStruct((NPAD2 * H,), jnp.float32),
                  jax.ShapeDtypeStruct((NPAD2 * H,), jnp.float32)),
        scratch_types=_prop_scratch(),
    )
    def k(hs_h, src_h, dst_h, out0, out1, src_b, dst_b, st_src,
          st_dstl, rows, acc_flat, wpr, gsem):
        cid = lax.axis_index("c")
        sid = lax.axis_index("s")

        @pl.when(cid == 0)
        def _():
            _owner_scan(hs_h, out0, src_h, dst_h, src_b, dst_b, st_src,
                        st_dstl, rows, acc_flat, wpr, gsem, sid, 0, epc)

        @pl.when(cid == 1)
        def _():
            _owner_scan(hs_h, out1, src_h, dst_h, src_b, dst_b, st_src,
                        st_dstl, rows, acc_flat, wpr, gsem, sid, epc, epc)

    return k(hs2, src, dst)


def _sc_edge_dot(x, srcg, dstg):
    e2 = srcg.shape[0] * G
    ngrp = srcg.shape[0]
    base_g = ngrp // NW
    extra = ngrp - base_g * NW

    KD = 3

    @functools.partial(
        pl.kernel,
        mesh=_sc_mesh(),
        compiler_params=pltpu.CompilerParams(needs_layout_passes=False,
                                             use_tc_tiling_on_sc=False),
        out_type=jax.ShapeDtypeStruct((e2,), jnp.float32),
        scratch_types=[
            pltpu.VMEM((KD, G), jnp.int32),
            pltpu.VMEM((KD, G), jnp.int32),
            pltpu.VMEM((KD, G, O), jnp.float32),
            pltpu.VMEM((KD, G, O), jnp.float32),
            pltpu.VMEM((G,), jnp.float32),
        ] + [pltpu.SemaphoreType.DMA] * (2 * KD),
    )
    def k(x_h, srcg_h, dstg_h, res, si, di, xs, xd, out_v, *sems):
        cid = lax.axis_index("c")
        sid = lax.axis_index("s")
        wid = sid * NC + cid

        zero = jnp.zeros((16,), jnp.float32)

        def issue(g, b):
            pltpu.sync_copy(srcg_h.at[g], si.at[b])
            pltpu.sync_copy(dstg_h.at[g], di.at[b])
            cp0 = pltpu.async_copy(x_h.at[si.at[b]], xs.at[b], sems[2 * b])
            cp1 = pltpu.async_copy(x_h.at[di.at[b]], xd.at[b],
                                   sems[2 * b + 1])
            return cp0, cp1

        def compute(g, b, cps):
            cps[0].wait()
            cps[1].wait()
            for z in range(G // 16):
                out_v[pl.ds(z * 16, 16)] = zero

            def edge(e, _):
                p = xs[b, e, pl.ds(0, 16)] * xd[b, e, pl.ds(0, 16)]
                for j in range(1, O // 16):
                    p = p + (xs[b, e, pl.ds(j * 16, 16)]
                             * xd[b, e, pl.ds(j * 16, 16)])
                eidx = jnp.broadcast_to(e, (16,)).astype(jnp.int32)
                plsc.addupdate_scatter(out_v, [eidx], p)
                return 0

            lax.fori_loop(0, G, edge, 0)
            pltpu.sync_copy(out_v, res.at[pl.ds(g * G, G)])

        def body(k_, _):
            gbase = wid + NW * KD * k_
            cps = [issue(gbase + NW * b, b) for b in range(KD)]
            for b in range(KD):
                compute(gbase + NW * b, b, cps[b])
            return 0

        lax.fori_loop(0, base_g // KD, body, 0)

        for r in range(base_g - (base_g // KD) * KD):
            g = wid + NW * ((base_g // KD) * KD + r)
            cps = issue(g, 0)
            compute(g, 0, cps)

        @pl.when(wid < extra)
        def _():
            g = wid + NW * base_g
            cps = issue(g, 0)
            compute(g, 0, cps)

    return k(x, srcg, dstg)


def _tc_dinv(part):
    part3 = part.reshape(NW, NPAD // 128, 128)

    def body(p_ref, o_ref):
        deg = jnp.sum(p_ref[...], axis=0) + 1.0
        o_ref[...] = lax.rsqrt(deg)

    out = pl.pallas_call(
        body,
        out_shape=jax.ShapeDtypeStruct((NPAD // 128, 128), jnp.float32),
    )(part3)
    return out.reshape(NPAD)


def _tc_mm_scale(x, w, dinv2, block_rows=2000):
    m, kdim = x.shape
    n = w.shape[1]

    def body(x_ref, w_ref, d_ref, o_ref):
        o_ref[...] = jnp.dot(x_ref[...], w_ref[...],
                             preferred_element_type=jnp.float32) * d_ref[...]

    return pl.pallas_call(
        body,
        grid=(m // block_rows,),
        in_specs=[
            pl.BlockSpec((block_rows, kdim), lambda i: (i, 0)),
            pl.BlockSpec((kdim, n), lambda i: (0, 0)),
            pl.BlockSpec((block_rows, 1), lambda i: (i, 0)),
        ],
        out_specs=pl.BlockSpec((block_rows, n), lambda i: (i, 0)),
        out_shape=jax.ShapeDtypeStruct((m, n), jnp.float32),
    )(x, w, dinv2)


def _tc_layer2(accA, hsA, accB, hsB, dinv2, b1, W2, block_rows=2000):
    m = accA.shape[0]

    def body(aA, hA, aB, hB, d_ref, b_ref, w_ref, o_ref):
        d = d_ref[...]
        tA = jax.nn.relu(d * (aA[...] + hA[...]) + b_ref[...])
        tB = jax.nn.relu(d * (aB[...] + hB[...]) + b_ref[...])
        oA = jnp.dot(tA, w_ref[...], preferred_element_type=jnp.float32) * d
        oB = jnp.dot(tB, w_ref[...], preferred_element_type=jnp.float32) * d
        o_ref[...] = jnp.concatenate([oA, oB], axis=1)

    return pl.pallas_call(
        body,
        grid=(m // block_rows,),
        in_specs=[
            pl.BlockSpec((block_rows, H), lambda i: (i, 0)),
            pl.BlockSpec((block_rows, H), lambda i: (i, 0)),
            pl.BlockSpec((block_rows, H), lambda i: (i, 0)),
            pl.BlockSpec((block_rows, H), lambda i: (i, 0)),
            pl.BlockSpec((block_rows, 1), lambda i: (i, 0)),
            pl.BlockSpec((1, H), lambda i: (0, 0)),
            pl.BlockSpec((H, O), lambda i: (0, 0)),
        ],
        out_specs=pl.BlockSpec((block_rows, 2 * O), lambda i: (i, 0)),
        out_shape=jax.ShapeDtypeStruct((m, 2 * O), jnp.float32),
    )(accA, hsA, accB, hsB, dinv2, b1.reshape(1, H), W2)


def _tc_final(acc0, acc1, hs2, dinv2, b2, Wm, bm, Wa, ba, block_rows=2000):
    m = hs2.shape[0]

    def body(a0, a1, h_ref, d_ref, b2_ref, wm_ref, bm_ref, wa_ref, ba_ref,
             x_ref, att_ref):
        d = d_ref[...]
        xc = d * (a0[...] + a1[...] + h_ref[...]) + b2_ref[...]
        x_ref[...] = jnp.dot(xc, wm_ref[...],
                             preferred_element_type=jnp.float32) + bm_ref[...]
        x2 = xc[:, O:]
        t = jnp.dot(x2, wa_ref[...],
                    preferred_element_type=jnp.float32) + ba_ref[...]
        tm = jnp.max(t, axis=1, keepdims=True)
        tt = t - tm
        att_ref[...] = tt - jnp.log(jnp.sum(jnp.exp(tt), axis=1,
                                            keepdims=True))

    b22 = jnp.concatenate([b2, b2]).reshape(1, 2 * O)
    return pl.pallas_call(
        body,
        grid=(m // block_rows,),
        in_specs=[
            pl.BlockSpec((block_rows, 2 * O), lambda i: (i, 0)),
            pl.BlockSpec((block_rows, 2 * O), lambda i: (i, 0)),
            pl.BlockSpec((block_rows, 2 * O), lambda i: (i, 0)),
            pl.BlockSpec((block_rows, 1), lambda i: (i, 0)),
            pl.BlockSpec((1, 2 * O), lambda i: (0, 0)),
            pl.BlockSpec((2 * O, O), lambda i: (0, 0)),
            pl.BlockSpec((1, O), lambda i: (0, 0)),
            pl.BlockSpec((O, C), lambda i: (0, 0)),
            pl.BlockSpec((1, C), lambda i: (0, 0)),
        ],
        out_specs=[
            pl.BlockSpec((block_rows, O), lambda i: (i, 0)),
            pl.BlockSpec((block_rows, C), lambda i: (i, 0)),
        ],
        out_shape=[
            jax.ShapeDtypeStruct((m, O), jnp.float32),
            jax.ShapeDtypeStruct((m, C), jnp.float32),
        ],
    )(acc0, acc1, hs2, dinv2, b22, Wm, bm.reshape(1, O), Wa,
      ba.reshape(1, C))


def kernel(x_A, x_B, train_pos_edge_index, pos_edge_index, neg_edge_index,
           W1, b1, W2, b2, Wm, bm, Wa, ba):
    src = train_pos_edge_index[0]
    dst = train_pos_edge_index[1]

    part = _sc_degree(dst)
    dinv = _tc_dinv(part)[:N]
    dinv2 = dinv[:, None]

    hsA = _tc_mm_scale(x_A, W1, dinv2)
    hsB = _tc_mm_scale(x_B, W1, dinv2)

    accA, accB = _sc_prop1(hsA, hsB, src, dst)
    accA = accA.reshape(NPAD2, H)[:N]
    accB = accB.reshape(NPAD2, H)[:N]

    hs2 = _tc_layer2(accA, hsA, accB, hsB, dinv2, b1, W2)

    acc0, acc1 = _sc_prop2(hs2, src, dst)
    acc0 = acc0.reshape(NPAD2, H)[:N]
    acc1 = acc1.reshape(NPAD2, H)[:N]

    x, att = _tc_final(acc0, acc1, hs2, dinv2, b2, Wm, bm, Wa, ba)

    tot = jnp.concatenate([pos_edge_index, neg_edge_index], axis=-1)
    e2 = tot.shape[1]
    res = _sc_edge_dot(x, tot[0].reshape(e2 // G, G), tot[1].reshape(e2 // G, G))
    return res, att


# speedup vs baseline: 1.2784x; 1.0000x over previous
import functools

import jax
import jax.numpy as jnp
from jax import lax
from jax.experimental import pallas as pl
from jax.experimental.pallas import tpu as pltpu
from jax.experimental.pallas import tpu_sc as plsc

N = 10000
NPAD = 10112
E = 160000
G = 128
H = 128
O = 64
C = 16

NC = 2
NS = 16
NW = NC * NS


def _sc_mesh():
    return plsc.VectorSubcoreMesh(core_axis_name="c", subcore_axis_name="s")


def _sc_degree(dst):
    ept = E // NW
    full = ept // 16
    tail = ept - full * 16

    @functools.partial(
        pl.kernel,
        mesh=_sc_mesh(),
        compiler_params=pltpu.CompilerParams(needs_layout_passes=False),
        out_type=jax.ShapeDtypeStruct((NW, NPAD), jnp.float32),
        scratch_types=[
            pltpu.VMEM((NPAD,), jnp.float32),
            pltpu.VMEM((ept + 16,), jnp.int32),
        ],
    )
    def k(dst_hbm, out_hbm, hist, idx):
        cid = lax.axis_index("c")
        sid = lax.axis_index("s")
        wid = sid * NC + cid

        zero = jnp.zeros((16,), jnp.float32)

        def zbody(i, _):
            hist[pl.ds(i * 16, 16)] = zero
            return 0

        lax.fori_loop(0, NPAD // 16, zbody, 0)

        pltpu.sync_copy(dst_hbm.at[pl.ds(wid * ept, ept)],
                        idx.at[pl.ds(0, ept)])

        ones = jnp.ones((16,), jnp.float32)

        def hbody(i, _):
            v = idx[pl.ds(i * 16, 16)]
            plsc.addupdate_scatter(hist, [v], ones)
            return 0

        lax.fori_loop(0, full, hbody, 0)

        if tail:
            lanes = lax.iota(jnp.int32, 16)
            tmask = lanes < tail
            v = idx[pl.ds(full * 16, 16)]
            v = jnp.where(tmask, v, 0)
            plsc.addupdate_scatter(hist, [v], ones, mask=tmask)

        pltpu.sync_copy(hist, out_hbm.at[wid])

    return k(dst)


OWN = 640
NPAD2 = 16 * OWN
ACC_ROWS = OWN + 8
ST_CAP = 272
EB = 1024


def _drain(hs, acc_flat, st_src, st_dstl, rows, gsem):
    pltpu.async_copy(hs.at[st_src.at[pl.ds(0, G)]], rows, gsem).wait()
    iota = lax.iota(jnp.int32, 16)

    def edge(e, _):
        base = plsc.load_gather(
            st_dstl, [jnp.broadcast_to(e, (16,)).astype(jnp.int32)])
        addr = base * H + iota
        for j in range(H // 16):
            plsc.addupdate_scatter(acc_flat, [addr + j * 16],
                                   rows[e, pl.ds(j * 16, 16)])
        return 0

    lax.fori_loop(0, G, edge, 0)


def _owner_scan(hs, out_flat, src_h, dst_h, src_b, dst_b, st_src, st_dstl,
                rows, acc_flat, wpr, gsem, sid, eoff, n_edges):
    iota = lax.iota(jnp.int32, 16)
    zero = jnp.zeros((16,), jnp.float32)

    def zb(i, _):
        acc_flat[pl.ds(i * 16, 16)] = zero
        return 0

    lax.fori_loop(0, (ACC_ROWS * H) // 16, zb, 0)
    wpr[0] = 0

    my_base = sid * OWN

    UNROLL = 8

    def block(i, _):
        wp = wpr[0]
        vals = []
        for u in range(UNROLL):
            c = i * UNROLL + u
            s16 = src_b[pl.ds(c * 16, 16)]
            d16 = dst_b[pl.ds(c * 16, 16)]
            m = jnp.right_shift(d16 * 52429, 25) == sid
            cnt = lax.reduce_sum(m.astype(jnp.int32), axes=(0,))
            vals.append((s16, d16, m, cnt))
        for s16, d16, m, cnt in vals:
            plsc.store_compressed(st_src.at[pl.ds(wp, 16)], s16, mask=m)
            plsc.store_compressed(st_dstl.at[pl.ds(wp, 16)], d16 - my_base,
                                  mask=m)
            wp = wp + cnt

        @pl.when(wp >= G)
        def _():
            _drain(hs, acc_flat, st_src, st_dstl, rows, gsem)
            for t in range(8):
                t0 = st_src[pl.ds(G + t * 16, 16)]
                t1 = st_dstl[pl.ds(G + t * 16, 16)]
                st_src[pl.ds(t * 16, 16)] = t0
                st_dstl[pl.ds(t * 16, 16)] = t1
            wpr[0] = wp - G

        @pl.when(wp < G)
        def _():
            wpr[0] = wp

        return 0

    n_full = n_edges // EB
    tail = n_edges - n_full * EB

    def load_chunk(l, nb):
        pltpu.sync_copy(src_h.at[pl.ds(eoff + l * EB, nb)],
                        src_b.at[pl.ds(0, nb)])
        pltpu.sync_copy(dst_h.at[pl.ds(eoff + l * EB, nb)],
                        dst_b.at[pl.ds(0, nb)])
        lax.fori_loop(0, nb // (16 * UNROLL), block, 0)

    def chunk(l, _):
        load_chunk(l, EB)
        return 0

    lax.fori_loop(0, n_full, chunk, 0)
    if tail:
        load_chunk(n_full, tail)

    wp = wpr[0]
    for p in range(G // 16):
        idx16 = wp + p * 16 + iota
        pm = idx16 < G
        plsc.store_scatter(st_src, [idx16], jnp.zeros((16,), jnp.int32),
                           mask=pm)
        plsc.store_scatter(st_dstl, [idx16],
                           jnp.full((16,), OWN, jnp.int32), mask=pm)
    _drain(hs, acc_flat, st_src, st_dstl, rows, gsem)

    pltpu.sync_copy(acc_flat.at[pl.ds(0, OWN * H)],
                    out_flat.at[pl.ds(sid * (OWN * H), OWN * H)])


def _prop_scratch():
    return [
        pltpu.VMEM((EB,), jnp.int32),
        pltpu.VMEM((EB,), jnp.int32),
        pltpu.VMEM((ST_CAP,), jnp.int32),
        pltpu.VMEM((ST_CAP,), jnp.int32),
        pltpu.VMEM((G, H), jnp.float32),
        pltpu.VMEM((ACC_ROWS * H,), jnp.float32),
        pltpu.SMEM((1,), jnp.int32),
        pltpu.SemaphoreType.DMA,
    ]


def _sc_prop1(hsA, hsB, src, dst):
    @functools.partial(
        pl.kernel,
        mesh=_sc_mesh(),
        compiler_params=pltpu.CompilerParams(needs_layout_passes=False),
        out_type=(jax.ShapeDtypeStruct((NPAD2 * H,), jnp.float32),
                  jax.ShapeDtypeStruct((NPAD2 * H,), jnp.float32)),
        scratch_types=_prop_scratch(),
    )
    def k(hsA_h, hsB_h, src_h, dst_h, outA, outB, src_b, dst_b, st_src,
          st_dstl, rows, acc_flat, wpr, gsem):
        cid = lax.axis_index("c")
        sid = lax.axis_index("s")

        @pl.when(cid == 0)
        def _():
            _owner_scan(hsA_h, outA, src_h, dst_h, src_b, dst_b, st_src,
                        st_dstl, rows, acc_flat, wpr, gsem, sid, 0, E)

        @pl.when(cid == 1)
        def _():
            _owner_scan(hsB_h, outB, src_h, dst_h, src_b, dst_b, st_src,
                        st_dstl, rows, acc_flat, wpr, gsem, sid, 0, E)

    return k(hsA, hsB, src, dst)


def _sc_prop2(hs2, src, dst):
    epc = E // NC

    @functools.partial(
        pl.kernel,
        mesh=_sc_mesh(),
        compiler_params=pltpu.CompilerParams(needs_layout_passes=False),
        out_type=(jax.ShapeDtypeStruct((NPAD2 * H,), jnp.float32),
                  jax.ShapeDtypeStruct((NPAD2 * H,), jnp.float32)),
        scratch_types=_prop_scratch(),
    )
    def k(hs_h, src_h, dst_h, out0, out1, src_b, dst_b, st_src,
          st_dstl, rows, acc_flat, wpr, gsem):
        cid = lax.axis_index("c")
        sid = lax.axis_index("s")

        @pl.when(cid == 0)
        def _():
            _owner_scan(hs_h, out0, src_h, dst_h, src_b, dst_b, st_src,
                        st_dstl, rows, acc_flat, wpr, gsem, sid, 0, epc)

        @pl.when(cid == 1)
        def _():
            _owner_scan(hs_h, out1, src_h, dst_h, src_b, dst_b, st_src,
                        st_dstl, rows, acc_flat, wpr, gsem, sid, epc, epc)

    return k(hs2, src, dst)


def _sc_edge_dot(x, srcg, dstg):
    e2 = srcg.shape[0] * G
    ngrp = srcg.shape[0]
    base_g = ngrp // NW
    extra = ngrp - base_g * NW

    KD = 3

    @functools.partial(
        pl.kernel,
        mesh=_sc_mesh(),
        compiler_params=pltpu.CompilerParams(needs_layout_passes=False,
                                             use_tc_tiling_on_sc=False),
        out_type=jax.ShapeDtypeStruct((e2,), jnp.float32),
        scratch_types=[
            pltpu.VMEM((KD, G), jnp.int32),
            pltpu.VMEM((KD, G), jnp.int32),
            pltpu.VMEM((KD, G, O), jnp.float32),
            pltpu.VMEM((KD, G, O), jnp.float32),
            pltpu.VMEM((G,), jnp.float32),
        ] + [pltpu.SemaphoreType.DMA] * (2 * KD),
    )
    def k(x_h, srcg_h, dstg_h, res, si, di, xs, xd, out_v, *sems):
        cid = lax.axis_index("c")
        sid = lax.axis_index("s")
        wid = sid * NC + cid

        zero = jnp.zeros((16,), jnp.float32)

        def issue(g, b):
            pltpu.sync_copy(srcg_h.at[g], si.at[b])
            pltpu.sync_copy(dstg_h.at[g], di.at[b])
            cp0 = pltpu.async_copy(x_h.at[si.at[b]], xs.at[b], sems[2 * b])
            cp1 = pltpu.async_copy(x_h.at[di.at[b]], xd.at[b],
                                   sems[2 * b + 1])
            return cp0, cp1

        def compute(g, b, cps):
            cps[0].wait()
            cps[1].wait()
            for z in range(G // 16):
                out_v[pl.ds(z * 16, 16)] = zero

            def edge(e, _):
                p = xs[b, e, pl.ds(0, 16)] * xd[b, e, pl.ds(0, 16)]
                for j in range(1, O // 16):
                    p = p + (xs[b, e, pl.ds(j * 16, 16)]
                             * xd[b, e, pl.ds(j * 16, 16)])
                eidx = jnp.broadcast_to(e, (16,)).astype(jnp.int32)
                plsc.addupdate_scatter(out_v, [eidx], p)
                return 0

            lax.fori_loop(0, G, edge, 0)
            pltpu.sync_copy(out_v, res.at[pl.ds(g * G, G)])

        def body(k_, _):
            gbase = wid + NW * KD * k_
            cps = [issue(gbase + NW * b, b) for b in range(KD)]
            for b in range(KD):
                compute(gbase + NW * b, b, cps[b])
            return 0

        lax.fori_loop(0, base_g // KD, body, 0)

        for r in range(base_g - (base_g // KD) * KD):
            g = wid + NW * ((base_g // KD) * KD + r)
            cps = issue(g, 0)
            compute(g, 0, cps)

        @pl.when(wid < extra)
        def _():
            g = wid + NW * base_g
            cps = issue(g, 0)
            compute(g, 0, cps)

    return k(x, srcg, dstg)


def _tc_dinv(part):
    part3 = part.reshape(NW, NPAD // 128, 128)

    def body(p_ref, o_ref):
        deg = jnp.sum(p_ref[...], axis=0) + 1.0
        o_ref[...] = lax.rsqrt(deg)

    out = pl.pallas_call(
        body,
        out_shape=jax.ShapeDtypeStruct((NPAD // 128, 128), jnp.float32),
    )(part3)
    return out.reshape(NPAD)


def _tc_mm_scale(x, w, dinv2, block_rows=2000):
    m, kdim = x.shape
    n = w.shape[1]

    def body(x_ref, w_ref, d_ref, o_ref):
        o_ref[...] = jnp.dot(x_ref[...], w_ref[...],
                             preferred_element_type=jnp.float32) * d_ref[...]

    return pl.pallas_call(
        body,
        grid=(m // block_rows,),
        in_specs=[
            pl.BlockSpec((block_rows, kdim), lambda i: (i, 0)),
            pl.BlockSpec((kdim, n), lambda i: (0, 0)),
            pl.BlockSpec((block_rows, 1), lambda i: (i, 0)),
        ],
        out_specs=pl.BlockSpec((block_rows, n), lambda i: (i, 0)),
        out_shape=jax.ShapeDtypeStruct((m, n), jnp.float32),
    )(x, w, dinv2)


def _tc_layer2(accA, hsA, accB, hsB, dinv2, b1, W2, block_rows=2000):
    m = accA.shape[0]

    def body(aA, hA, aB, hB, d_ref, b_ref, w_ref, o_ref):
        d = d_ref[...]
        tA = jax.nn.relu(d * (aA[...] + hA[...]) + b_ref[...])
        tB = jax.nn.relu(d * (aB[...] + hB[...]) + b_ref[...])
        oA = jnp.dot(tA, w_ref[...], preferred_element_type=jnp.float32) * d
        oB = jnp.dot(tB, w_ref[...], preferred_element_type=jnp.float32) * d
        o_ref[...] = jnp.concatenate([oA, oB], axis=1)

    return pl.pallas_call(
        body,
        grid=(m // block_rows,),
        in_specs=[
            pl.BlockSpec((block_rows, H), lambda i: (i, 0)),
            pl.BlockSpec((block_rows, H), lambda i: (i, 0)),
            pl.BlockSpec((block_rows, H), lambda i: (i, 0)),
            pl.BlockSpec((block_rows, H), lambda i: (i, 0)),
            pl.BlockSpec((block_rows, 1), lambda i: (i, 0)),
            pl.BlockSpec((1, H), lambda i: (0, 0)),
            pl.BlockSpec((H, O), lambda i: (0, 0)),
        ],
        out_specs=pl.BlockSpec((block_rows, 2 * O), lambda i: (i, 0)),
        out_shape=jax.ShapeDtypeStruct((m, 2 * O), jnp.float32),
    )(accA, hsA, accB, hsB, dinv2, b1.reshape(1, H), W2)


def _tc_final(acc0, acc1, hs2, dinv2, b2, Wm, bm, Wa, ba, block_rows=2000):
    m = hs2.shape[0]

    def body(a0, a1, h_ref, d_ref, b2_ref, wm_ref, bm_ref, wa_ref, ba_ref,
             x_ref, att_ref):
        d = d_ref[...]
        xc = d * (a0[...] + a1[...] + h_ref[...]) + b2_ref[...]
        x_ref[...] = jnp.dot(xc, wm_ref[...],
                             preferred_element_type=jnp.float32) + bm_ref[...]
        x2 = xc[:, O:]
        t = jnp.dot(x2, wa_ref[...],
                    preferred_element_type=jnp.float32) + ba_ref[...]
        tm = jnp.max(t, axis=1, keepdims=True)
        tt = t - tm
        att_ref[...] = tt - jnp.log(jnp.sum(jnp.exp(tt), axis=1,
                                            keepdims=True))

    b22 = jnp.concatenate([b2, b2]).reshape(1, 2 * O)
    return pl.pallas_call(
        body,
        grid=(m // block_rows,),
        in_specs=[
            pl.BlockSpec((block_rows, 2 * O), lambda i: (i, 0)),
            pl.BlockSpec((block_rows, 2 * O), lambda i: (i, 0)),
            pl.BlockSpec((block_rows, 2 * O), lambda i: (i, 0)),
            pl.BlockSpec((block_rows, 1), lambda i: (i, 0)),
            pl.BlockSpec((1, 2 * O), lambda i: (0, 0)),
            pl.BlockSpec((2 * O, O), lambda i: (0, 0)),
            pl.BlockSpec((1, O), lambda i: (0, 0)),
            pl.BlockSpec((O, C), lambda i: (0, 0)),
            pl.BlockSpec((1, C), lambda i: (0, 0)),
        ],
        out_specs=[
            pl.BlockSpec((block_rows, O), lambda i: (i, 0)),
            pl.BlockSpec((block_rows, C), lambda i: (i, 0)),
        ],
        out_shape=[
            jax.ShapeDtypeStruct((m, O), jnp.float32),
            jax.ShapeDtypeStruct((m, C), jnp.float32),
        ],
    )(acc0, acc1, hs2, dinv2, b22, Wm, bm.reshape(1, O), Wa,
      ba.reshape(1, C))


def kernel(x_A, x_B, train_pos_edge_index, pos_edge_index, neg_edge_index,
           W1, b1, W2, b2, Wm, bm, Wa, ba):
    src = train_pos_edge_index[0]
    dst = train_pos_edge_index[1]

    part = _sc_degree(dst)
    dinv = _tc_dinv(part)[:N]
    dinv2 = dinv[:, None]

    hsA = _tc_mm_scale(x_A, W1, dinv2)
    hsB = _tc_mm_scale(x_B, W1, dinv2)

    accA, accB = _sc_prop1(hsA, hsB, src, dst)
    accA = accA.reshape(NPAD2, H)[:N]
    accB = accB.reshape(NPAD2, H)[:N]

    hs2 = _tc_layer2(accA, hsA, accB, hsB, dinv2, b1, W2)

    acc0, acc1 = _sc_prop2(hs2, src, dst)
    acc0 = acc0.reshape(NPAD2, H)[:N]
    acc1 = acc1.reshape(NPAD2, H)[:N]

    x, att = _tc_final(acc0, acc1, hs2, dinv2, b2, Wm, bm, Wa, ba)

    tot = jnp.concatenate([pos_edge_index, neg_edge_index], axis=-1)
    e2 = tot.shape[1]
    res = _sc_edge_dot(x, tot[0].reshape(e2 // G, G), tot[1].reshape(e2 // G, G))
    return res, att
